# baseline (speedup 1.0000x reference)
import functools

import jax
import jax.numpy as jnp
from jax import lax
from jax.experimental import pallas as pl
from jax.experimental.pallas import tpu as pltpu
from jax.experimental.pallas import tpu_sc as plsc

F32 = jnp.float32
BF16 = jnp.bfloat16
I32 = jnp.int32

H = 128
NPAD = 10240
EPAD = 327680
XPAD = 32768
NBLK = 2048
EBLK = 4096
NWORK = 32
EG = 80
NROWS_PER_SUB = NPAD // 16

@functools.cache
def _mesh():
    return plsc.VectorSubcoreMesh(core_axis_name="c", subcore_axis_name="s",
                                  num_cores=2, num_subcores=16)


def _sc_gather16(table, idx2d):
    @functools.partial(
        pl.kernel,
        out_type=jax.ShapeDtypeStruct((XPAD, 128), F32),
        mesh=_mesh(),
        scratch_types=[
            pltpu.VMEM((8, 128), I32),
            pltpu.VMEM((128, 128), F32),
            pltpu.SemaphoreType.DMA,
        ],
    )
    def k(tab_hbm, idx_hbm, out_hbm, idx_v, buf, sem):
        wid = lax.axis_index("s") * 2 + lax.axis_index("c")
        pltpu.sync_copy(idx_hbm.at[pl.ds(wid * 8, 8)], idx_v)

        @pl.loop(0, 8)
        def _(g):
            pltpu.async_copy(tab_hbm.at[idx_v.at[g]], buf, sem).wait()
            pltpu.sync_copy(buf, out_hbm.at[pl.ds(wid * 1024 + g * 128, 128)])

    return k(table, idx2d)


def _sc_edge_gather(tab_a, tab_b, src2d, dst2d, eg):
    ne = NWORK * eg * 128
    @functools.partial(
        pl.kernel,
        out_type=jax.ShapeDtypeStruct((ne, H), F32),
        mesh=_mesh(),
        scratch_types=[
            pltpu.VMEM((eg, 128), I32),
            pltpu.VMEM((eg, 128), I32),
            pltpu.VMEM((128, H), F32),
            pltpu.VMEM((128, H), F32),
            pltpu.VMEM((128, H), F32),
            pltpu.VMEM((128, H), F32),
        ] + [pltpu.SemaphoreType.DMA] * 8,
    )
    def k(a_hbm, b_hbm, s_hbm, d_hbm, o_hbm,
          idxa, idxb, buf0, buf1, buf2, buf3,
          sg0, sg1, sg2, sg3, sw0, sw1, sw2, sw3):
        wid = lax.axis_index("s") * 2 + lax.axis_index("c")
        pltpu.sync_copy(s_hbm.at[pl.ds(wid * eg, eg)], idxa)
        pltpu.sync_copy(d_hbm.at[pl.ds(wid * eg, eg)], idxb)
        bufs = (buf0, buf1, buf2, buf3)
        sg = (sg0, sg1, sg2, sg3)
        sw = (sw0, sw1, sw2, sw3)

        def issue_ga(sl, g):
            pltpu.async_copy(a_hbm.at[idxa.at[g]], bufs[sl], sg[sl])

        def wait_ga(sl, g):
            pltpu.make_async_copy(a_hbm.at[idxa.at[g]], bufs[sl],
                                  sg[sl]).wait()

        def issue_gb(sl, g):
            pltpu.async_copy(b_hbm.at[idxb.at[g]], bufs[sl], sg[sl],
                             add=True)

        def wait_gb(sl, g):
            pltpu.make_async_copy(b_hbm.at[idxb.at[g]], bufs[sl],
                                  sg[sl]).wait()

        def issue_w(sl, g):
            row0 = wid * (eg * 128) + g * 128
            pltpu.async_copy(bufs[sl], o_hbm.at[pl.ds(row0, 128)], sw[sl])

        def wait_w(sl):
            pltpu.make_async_copy(bufs[sl], o_hbm.at[pl.ds(0, 128)],
                                  sw[sl]).wait()

        @pl.loop(0, eg, step=4)
        def _(g):
            for sl in range(4):
                @pl.when(g >= 4)
                def _():
                    wait_w(sl)

                issue_ga(sl, g + sl)
            for sl in range(4):
                wait_ga(sl, g + sl)
                issue_gb(sl, g + sl)
            for sl in range(4):
                wait_gb(sl, g + sl)
                issue_w(sl, g + sl)

        for sl in range(4):
            wait_w(sl)

    return k(tab_a, tab_b, src2d, dst2d)


def _sc_scatter_add(msgs, dst2d, zeros_tab, eg):
    @functools.partial(
        pl.kernel,
        out_type=jax.ShapeDtypeStruct((2, NPAD, H), F32),
        mesh=_mesh(),
        scratch_types=[
            pltpu.VMEM((eg, 128), I32),
            pltpu.VMEM((128, H), F32),
            pltpu.VMEM((128, H), F32),
            pltpu.VMEM_SHARED((NPAD, H), F32),
        ] + [pltpu.SemaphoreType.DMA] * 4,
    )
    def k(m_hbm, d_hbm, z_hbm, o_hbm, idxd, bufm0, bufm1, spm,
          sr0, sr1, ss0, ss1):
        cid = lax.axis_index("c")
        sid = lax.axis_index("s")
        wid = sid * 2 + cid
        pltpu.sync_copy(z_hbm.at[pl.ds(sid * NROWS_PER_SUB, NROWS_PER_SUB)],
                        spm.at[pl.ds(sid * NROWS_PER_SUB, NROWS_PER_SUB)])
        plsc.subcore_barrier()
        pltpu.sync_copy(d_hbm.at[pl.ds(wid * eg, eg)], idxd)
        bufm = (bufm0, bufm1)
        sr = (sr0, sr1)
        ss = (ss0, ss1)

        def issue_read(sl, g):
            pltpu.async_copy(m_hbm.at[pl.ds(wid * (eg * 128) + g * 128, 128)],
                             bufm[sl], sr[sl])

        def wait_read(sl):
            pltpu.make_async_copy(m_hbm.at[pl.ds(0, 128)], bufm[sl],
                                  sr[sl]).wait()

        def issue_scat(sl, g):
            pltpu.async_copy(bufm[sl], spm.at[idxd.at[g]], ss[sl], add=True)

        def wait_scat(sl, g):
            pltpu.make_async_copy(bufm[sl], spm.at[idxd.at[g]],
                                  ss[sl]).wait()

        @pl.loop(0, eg, step=2)
        def _(g):
            for sl in range(2):
                @pl.when(g >= 2)
                def _():
                    wait_scat(sl, g + sl - 2)

                issue_read(sl, g + sl)
            for sl in range(2):
                wait_read(sl)
                issue_scat(sl, g + sl)

        for sl in range(2):
            wait_scat(sl, eg + sl - 2)
        plsc.subcore_barrier()
        pltpu.sync_copy(spm.at[pl.ds(sid * NROWS_PER_SUB, NROWS_PER_SUB)],
                        o_hbm.at[cid, pl.ds(sid * NROWS_PER_SUB, NROWS_PER_SUB)])

    return k(msgs, dst2d, zeros_tab)


def _dot(a, b):
    return jnp.dot(a, b, preferred_element_type=F32)


def _tc_pool_pos(pos_pad, batch_lanes):
    def body(pos_ref, bl_ref, pooled_ref, counts_ref):
        i = pl.program_id(0)
        brow = bl_ref[0][0:1, :]
        oht = (lax.broadcasted_iota(I32, (128, NBLK), 0) == brow).astype(F32)
        p = _dot(oht, pos_ref[...])
        c = _dot(oht, jnp.ones((NBLK, 128), F32))

        @pl.when(i == 0)
        def _():
            pooled_ref[...] = jnp.zeros_like(pooled_ref)
            counts_ref[...] = jnp.zeros_like(counts_ref)

        pooled_ref[...] += p
        counts_ref[...] += c

    return pl.pallas_call(
        body,
        grid=(NPAD // NBLK,),
        in_specs=[
            pl.BlockSpec((NBLK, 128), lambda i: (i, 0)),
            pl.BlockSpec((1, 8, NBLK), lambda i: (i, 0, 0)),
        ],
        out_specs=[
            pl.BlockSpec((128, 128), lambda i: (0, 0)),
            pl.BlockSpec((128, 128), lambda i: (0, 0)),
        ],
        out_shape=[jax.ShapeDtypeStruct((128, 128), F32),
                   jax.ShapeDtypeStruct((128, 128), F32)],
    )(pos_pad, batch_lanes)


def _tc_locmean(pos_pad, batchcol, pooled, countsm):
    def body(pos_ref, bc_ref, pooled_ref, counts_ref, out_ref):
        cent = pooled_ref[...] / jnp.maximum(counts_ref[...], 1.0)
        oh = (bc_ref[...] ==
              lax.broadcasted_iota(I32, (NBLK, 128), 1)).astype(F32)
        out_ref[...] = pos_ref[...] - _dot(oh, cent)

    return pl.pallas_call(
        body,
        grid=(NPAD // NBLK,),
        in_specs=[
            pl.BlockSpec((NBLK, 128), lambda i: (i, 0)),
            pl.BlockSpec((NBLK, 128), lambda i: (i, 0)),
            pl.BlockSpec((128, 128), lambda i: (0, 0)),
            pl.BlockSpec((128, 128), lambda i: (0, 0)),
        ],
        out_specs=pl.BlockSpec((NBLK, 128), lambda i: (i, 0)),
        out_shape=jax.ShapeDtypeStruct((NPAD, 128), F32),
    )(pos_pad, batchcol, pooled, countsm)


def _tc_embed(gpad, ntcol, wemb, wm1, bm1row, n_real):
    def body(g_ref, nt_ref, we_ref, wm1_ref, bm1_ref, x_ref, a_ref, b_ref):
        g = g_ref[...]
        e0 = _dot(g, we_ref[0])
        e1 = _dot(g, we_ref[1])
        e2 = _dot(g, we_ref[2])
        nt = nt_ref[...]
        x = jnp.where(nt == 0, e0, jnp.where(nt == 1, e1, e2))
        i = pl.program_id(0)
        row = lax.broadcasted_iota(I32, (NBLK, 128), 0) + i * NBLK
        x = jnp.where(row < n_real, x, 0.0)
        x_ref[...] = x
        a_ref[...] = _dot(x, wm1_ref[0:H, :]) + bm1_ref[0:1, :]
        b_ref[...] = _dot(x, wm1_ref[H:2 * H, :])

    return pl.pallas_call(
        body,
        grid=(NPAD // NBLK,),
        in_specs=[
            pl.BlockSpec((NBLK, 128), lambda i: (i, 0)),
            pl.BlockSpec((NBLK, 128), lambda i: (i, 0)),
            pl.BlockSpec((3, 128, 128), lambda i: (0, 0, 0)),
            pl.BlockSpec((2 * H, H), lambda i: (0, 0)),
            pl.BlockSpec((8, 128), lambda i: (0, 0)),
        ],
        out_specs=[pl.BlockSpec((NBLK, 128), lambda i: (i, 0))] * 3,
        out_shape=[jax.ShapeDtypeStruct((NPAD, 128), F32)] * 3,
    )(gpad, ntcol, wemb, wm1, bm1row)


def _tc_edge_mlp(grows, wm2, bm2row):
    def body(g_ref, w2_ref, b2_ref, out_ref):
        gsum = jnp.maximum(g_ref[...], 0.0)
        out_ref[...] = jnp.maximum(_dot(gsum, w2_ref[...]) + b2_ref[0:1, :],
                                   0.0)

    ne = grows.shape[0]
    return pl.pallas_call(
        body,
        grid=(ne // EBLK,),
        in_specs=[
            pl.BlockSpec((EBLK, H), lambda i: (i, 0)),
            pl.BlockSpec((H, H), lambda i: (0, 0)),
            pl.BlockSpec((8, 128), lambda i: (0, 0)),
        ],
        out_specs=pl.BlockSpec((EBLK, H), lambda i: (i, 0)),
        out_shape=jax.ShapeDtypeStruct((ne, H), F32),
    )(grows, wm2, bm2row)


def _tc_update(x, aggp0, aggp1, wu1, bu1row, wu2, bu2row, wm1n, bm1nrow,
               n_real):
    def body(x_ref, p_ref, q_ref, wu1_ref, bu1_ref, wu2_ref, bu2_ref,
             wm1_ref, bm1_ref, xo_ref, ao_ref, bo_ref):
        agg = p_ref[0] + p_ref[1] + q_ref[0] + q_ref[1]
        x_in = x_ref[...]
        h = jnp.maximum(_dot(x_in, wu1_ref[0:H, :]) +
                        _dot(agg, wu1_ref[H:2 * H, :]) + bu1_ref[0:1, :], 0.0)
        xn = _dot(h, wu2_ref[...]) + bu2_ref[0:1, :]
        i = pl.program_id(0)
        row = lax.broadcasted_iota(I32, (NBLK, 128), 0) + i * NBLK
        xn = jnp.where(row < n_real, xn, 0.0)
        xo_ref[...] = xn
        ao_ref[...] = _dot(xn, wm1_ref[0:H, :]) + bm1_ref[0:1, :]
        bo_ref[...] = _dot(xn, wm1_ref[H:2 * H, :])

    return pl.pallas_call(
        body,
        grid=(NPAD // NBLK,),
        in_specs=[
            pl.BlockSpec((NBLK, 128), lambda i: (i, 0)),
            pl.BlockSpec((2, NBLK, 128), lambda i: (0, i, 0)),
            pl.BlockSpec((2, NBLK, 128), lambda i: (0, i, 0)),
            pl.BlockSpec((2 * H, H), lambda i: (0, 0)),
            pl.BlockSpec((8, 128), lambda i: (0, 0)),
            pl.BlockSpec((H, H), lambda i: (0, 0)),
            pl.BlockSpec((8, 128), lambda i: (0, 0)),
            pl.BlockSpec((2 * H, H), lambda i: (0, 0)),
            pl.BlockSpec((8, 128), lambda i: (0, 0)),
        ],
        out_specs=[pl.BlockSpec((NBLK, 128), lambda i: (i, 0))] * 3,
        out_shape=[jax.ShapeDtypeStruct((NPAD, 128), F32)] * 3,
    )(x, aggp0, aggp1, wu1, bu1row, wu2, bu2row, wm1n, bm1nrow)


def _tc_update_final(x, aggp0, aggp1, wu1, bu1row, wu2, bu2row, batch_lanes,
                     n_real):
    def body(x_ref, p_ref, q_ref, wu1_ref, bu1_ref, wu2_ref, bu2_ref, bl_ref,
             pooled_ref):
        agg = p_ref[0] + p_ref[1] + q_ref[0] + q_ref[1]
        x_in = x_ref[...]
        h = jnp.maximum(_dot(x_in, wu1_ref[0:H, :]) +
                        _dot(agg, wu1_ref[H:2 * H, :]) + bu1_ref[0:1, :], 0.0)
        xn = _dot(h, wu2_ref[...]) + bu2_ref[0:1, :]
        i = pl.program_id(0)
        row = lax.broadcasted_iota(I32, (NBLK, 128), 0) + i * NBLK
        xn = jnp.where(row < n_real, xn, 0.0)
        brow = bl_ref[0][0:1, :]
        oht = (lax.broadcasted_iota(I32, (128, NBLK), 0) == brow).astype(F32)
        p = _dot(oht, xn)

        @pl.when(i == 0)
        def _():
            pooled_ref[...] = jnp.zeros_like(pooled_ref)

        pooled_ref[...] += p

    return pl.pallas_call(
        body,
        grid=(NPAD // NBLK,),
        in_specs=[
            pl.BlockSpec((NBLK, 128), lambda i: (i, 0)),
            pl.BlockSpec((2, NBLK, 128), lambda i: (0, i, 0)),
            pl.BlockSpec((2, NBLK, 128), lambda i: (0, i, 0)),
            pl.BlockSpec((2 * H, H), lambda i: (0, 0)),
            pl.BlockSpec((8, 128), lambda i: (0, 0)),
            pl.BlockSpec((H, H), lambda i: (0, 0)),
            pl.BlockSpec((8, 128), lambda i: (0, 0)),
            pl.BlockSpec((1, 8, NBLK), lambda i: (i, 0, 0)),
        ],
        out_specs=pl.BlockSpec((128, 128), lambda i: (0, 0)),
        out_shape=jax.ShapeDtypeStruct((128, 128), F32),
    )(x, aggp0, aggp1, wu1, bu1row, wu2, bu2row, batch_lanes)


def _tc_head(pooled, countsm, wp_pad, bprow, ypad, n_graphs):
    def body(pooled_ref, counts_ref, wp_ref, bp_ref, y_ref,
             loss_ref, acc_ref, mloss_ref):
        pm = pooled_ref[...] / jnp.maximum(counts_ref[...], 1.0)
        out = _dot(pm, wp_ref[...]) + bp_ref[0:1, :]
        lane = lax.broadcasted_iota(I32, (128, 128), 1)
        row = lax.broadcasted_iota(I32, (128, 128), 0)
        z = jnp.where(lane < 2, out, -1e30)
        m = jnp.max(z, axis=1, keepdims=True)
        s = jnp.sum(jnp.where(lane < 2, jnp.exp(z - m), 0.0), axis=1,
                    keepdims=True)
        lse = m + jnp.log(s)
        y = y_ref[...]
        lossv = -jnp.sum(jnp.where(lane < 2, y * (out - lse), 0.0), axis=1,
                         keepdims=True)
        o0 = out[:, 0:1]
        o1 = out[:, 1:2]
        y0 = y[:, 0:1]
        y1 = y[:, 1:2]
        accv = ((o1 > o0) == (y1 > y0)).astype(F32)
        loss_ref[...] = jnp.broadcast_to(lossv, (128, 128))
        acc_ref[...] = jnp.broadcast_to(accv, (128, 128))
        total = jnp.sum(jnp.where(row[:, 0:1] < n_graphs, lossv, 0.0))
        mloss_ref[...] = jnp.full((8, 128), total / n_graphs, F32)

    return pl.pallas_call(
        body,
        grid=(1,),
        in_specs=[pl.BlockSpec((128, 128), lambda i: (0, 0))] * 5,
        out_specs=[
            pl.BlockSpec((128, 128), lambda i: (0, 0)),
            pl.BlockSpec((128, 128), lambda i: (0, 0)),
            pl.BlockSpec((8, 128), lambda i: (0, 0)),
        ],
        out_shape=[jax.ShapeDtypeStruct((128, 128), F32),
                   jax.ShapeDtypeStruct((128, 128), F32),
                   jax.ShapeDtypeStruct((8, 128), F32)],
    )(pooled, countsm, wp_pad, bprow, ypad)


def _rowpad(b):
    return jnp.broadcast_to(b[None, :], (8, 128)).astype(F32)


def kernel(pos, y, params, node_types, x_ind, batch, edge_index):
    n = pos.shape[0]
    e = edge_index.shape[1]
    nb = y.shape[0]

    pos_pad = jnp.zeros((NPAD, 128), F32).at[:n, :3].set(pos.astype(F32))
    batch_pad = jnp.concatenate(
        [batch.astype(I32), jnp.full((NPAD - n,), nb, I32)])
    batchcol = jnp.broadcast_to(batch_pad[:, None], (NPAD, 128))
    batch_lanes = jnp.broadcast_to(
        batch_pad.reshape(NPAD // NBLK, 1, NBLK), (NPAD // NBLK, 8, NBLK))
    nt_pad = jnp.concatenate(
        [node_types.astype(I32), jnp.zeros((NPAD - n,), I32)])
    ntcol = jnp.broadcast_to(nt_pad[:, None], (NPAD, 128))

    src = edge_index[0].astype(I32)
    dst = edge_index[1].astype(I32)
    epad_ids = jnp.arange(EPAD - e, dtype=I32)
    src2d = jnp.concatenate([src, epad_ids % n]).reshape(-1, 128)
    dst2d = jnp.concatenate([dst, n + epad_ids % (NPAD - n)]).reshape(-1, 128)

    xif = x_ind.reshape(-1).astype(I32)
    xif2d = jnp.concatenate(
        [xif, jnp.arange(XPAD - 3 * n, dtype=I32) % n]).reshape(-1, 128)

    zeros_tab = jnp.zeros((NPAD, H), F32)

    wemb = jnp.zeros((3, 128, 128), F32)
    for i in range(3):
        wi = params['emb'][i]['W']
        for j in range(i + 1):
            wemb = wemb.at[i, 16 * j:16 * j + 3, :].set(wi[3 * j:3 * j + 3, :])
        wemb = wemb.at[i, 127, :].set(params['emb'][i]['b'])

    wp_pad = jnp.zeros((128, 128), F32).at[:, :2].set(params['proj']['W'])
    bprow = jnp.zeros((8, 128), F32).at[0, :2].set(params['proj']['b'])
    bprow = jnp.broadcast_to(bprow[0:1, :], (8, 128))
    ypad = jnp.zeros((128, 128), F32).at[:nb, :2].set(y.astype(F32))

    layers = params['layers']

    pooled_pos, countsm = _tc_pool_pos(pos_pad, batch_lanes)
    locmean = _tc_locmean(pos_pad, batchcol, pooled_pos, countsm)

    g = _sc_gather16(locmean, xif2d)
    g48 = g[:3 * n].reshape(n, 3, 128)[:, :, :16].reshape(n, 48)
    gpad = jnp.zeros((NPAD, 128), F32).at[:n, :48].set(g48)
    gpad = gpad.at[:, 127].set(1.0)

    lp = layers[0]
    x, ta, tb = _tc_embed(gpad, ntcol, wemb, lp['Wm1'], _rowpad(lp['bm1']), n)

    hg = src2d.shape[0] // 2
    egh = hg // NWORK
    s2d = (src2d[:hg], src2d[hg:])
    d2d = (dst2d[:hg], dst2d[hg:])
    for l in range(3):
        lp = layers[l]
        bm2r = _rowpad(lp['bm2'])
        g0 = _sc_edge_gather(ta, tb, s2d[0], d2d[0], egh)
        g1 = _sc_edge_gather(ta, tb, s2d[1], d2d[1], egh)
        m0 = _tc_edge_mlp(g0, lp['Wm2'], bm2r)
        m1 = _tc_edge_mlp(g1, lp['Wm2'], bm2r)
        aggp0 = _sc_scatter_add(m0, d2d[0], zeros_tab, egh)
        aggp1 = _sc_scatter_add(m1, d2d[1], zeros_tab, egh)
        if l < 2:
            lpn = layers[l + 1]
            x, ta, tb = _tc_update(
                x, aggp0, aggp1, lp['Wu1'], _rowpad(lp['bu1']), lp['Wu2'],
                _rowpad(lp['bu2']), lpn['Wm1'], _rowpad(lpn['bm1']), n)
        else:
            pooled_x = _tc_update_final(
                x, aggp0, aggp1, lp['Wu1'], _rowpad(lp['bu1']), lp['Wu2'],
                _rowpad(lp['bu2']), batch_lanes, n)

    loss128, acc128, mloss = _tc_head(pooled_x, countsm, wp_pad, bprow, ypad,
                                      nb)

    loss = loss128[:nb, 0]
    acc = acc128[:nb, 0]
    backprop_loss = mloss[0, 0].reshape(())
    return backprop_loss, loss, acc


# speedup vs baseline: 1.0539x; 1.0539x over previous
import functools

import jax
import jax.numpy as jnp
from jax import lax
from jax.experimental import pallas as pl
from jax.experimental.pallas import tpu as pltpu
from jax.experimental.pallas import tpu_sc as plsc

F32 = jnp.float32
BF16 = jnp.bfloat16
I32 = jnp.int32

H = 128
NPAD = 10240
EPAD = 327680
XPAD = 32768
NBLK = 2048
EBLK = 4096
NWORK = 32
EG = 80
NROWS_PER_SUB = NPAD // 16

@functools.cache
def _mesh():
    return plsc.VectorSubcoreMesh(core_axis_name="c", subcore_axis_name="s",
                                  num_cores=2, num_subcores=16)


def _sc_gather16(table, idx2d):
    @functools.partial(
        pl.kernel,
        out_type=jax.ShapeDtypeStruct((XPAD, 128), F32),
        mesh=_mesh(),
        scratch_types=[
            pltpu.VMEM((8, 128), I32),
            pltpu.VMEM((128, 128), F32),
            pltpu.SemaphoreType.DMA,
        ],
    )
    def k(tab_hbm, idx_hbm, out_hbm, idx_v, buf, sem):
        wid = lax.axis_index("s") * 2 + lax.axis_index("c")
        pltpu.sync_copy(idx_hbm.at[pl.ds(wid * 8, 8)], idx_v)

        @pl.loop(0, 8)
        def _(g):
            pltpu.async_copy(tab_hbm.at[idx_v.at[g]], buf, sem).wait()
            pltpu.sync_copy(buf, out_hbm.at[pl.ds(wid * 1024 + g * 128, 128)])

    return k(table, idx2d)


def _sc_edge_gather(tab_a, tab_b, src2d, dst2d, eg):
    ne = NWORK * eg * 128
    @functools.partial(
        pl.kernel,
        out_type=jax.ShapeDtypeStruct((ne, H), F32),
        mesh=_mesh(),
        scratch_types=[
            pltpu.VMEM((eg, 128), I32),
            pltpu.VMEM((eg, 128), I32),
            pltpu.VMEM((128, H), F32),
            pltpu.VMEM((128, H), F32),
            pltpu.VMEM((128, H), F32),
            pltpu.VMEM((128, H), F32),
        ] + [pltpu.SemaphoreType.DMA] * 8,
    )
    def k(a_hbm, b_hbm, s_hbm, d_hbm, o_hbm,
          idxa, idxb, buf0, buf1, buf2, buf3,
          sg0, sg1, sg2, sg3, sw0, sw1, sw2, sw3):
        wid = lax.axis_index("s") * 2 + lax.axis_index("c")
        pltpu.sync_copy(s_hbm.at[pl.ds(wid * eg, eg)], idxa)
        pltpu.sync_copy(d_hbm.at[pl.ds(wid * eg, eg)], idxb)
        bufs = (buf0, buf1, buf2, buf3)
        sg = (sg0, sg1, sg2, sg3)
        sw = (sw0, sw1, sw2, sw3)

        def issue_ga(sl, g):
            pltpu.async_copy(a_hbm.at[idxa.at[g]], bufs[sl], sg[sl])

        def wait_ga(sl, g):
            pltpu.make_async_copy(a_hbm.at[idxa.at[g]], bufs[sl],
                                  sg[sl]).wait()

        def issue_gb(sl, g):
            pltpu.async_copy(b_hbm.at[idxb.at[g]], bufs[sl], sg[sl],
                             add=True)

        def wait_gb(sl, g):
            pltpu.make_async_copy(b_hbm.at[idxb.at[g]], bufs[sl],
                                  sg[sl]).wait()

        def issue_w(sl, g):
            row0 = wid * (eg * 128) + g * 128
            pltpu.async_copy(bufs[sl], o_hbm.at[pl.ds(row0, 128)], sw[sl])

        def wait_w(sl):
            pltpu.make_async_copy(bufs[sl], o_hbm.at[pl.ds(0, 128)],
                                  sw[sl]).wait()

        @pl.loop(0, eg, step=4)
        def _(g):
            for sl in range(4):
                @pl.when(g >= 4)
                def _():
                    wait_w(sl)

                issue_ga(sl, g + sl)
            for sl in range(4):
                wait_ga(sl, g + sl)
                issue_gb(sl, g + sl)
            for sl in range(4):
                wait_gb(sl, g + sl)
                issue_w(sl, g + sl)

        for sl in range(4):
            wait_w(sl)

    return k(tab_a, tab_b, src2d, dst2d)


def _sc_scatter_add(msgs, dst2d, zeros_tab, eg):
    @functools.partial(
        pl.kernel,
        out_type=jax.ShapeDtypeStruct((2, NPAD, H), F32),
        mesh=_mesh(),
        scratch_types=[
            pltpu.VMEM((eg, 128), I32),
            pltpu.VMEM((128, H), F32),
            pltpu.VMEM((128, H), F32),
            pltpu.VMEM_SHARED((NPAD, H), F32),
        ] + [pltpu.SemaphoreType.DMA] * 4,
    )
    def k(m_hbm, d_hbm, z_hbm, o_hbm, idxd, bufm0, bufm1, spm,
          sr0, sr1, ss0, ss1):
        cid = lax.axis_index("c")
        sid = lax.axis_index("s")
        wid = sid * 2 + cid
        pltpu.sync_copy(z_hbm.at[pl.ds(sid * NROWS_PER_SUB, NROWS_PER_SUB)],
                        spm.at[pl.ds(sid * NROWS_PER_SUB, NROWS_PER_SUB)])
        plsc.subcore_barrier()
        pltpu.sync_copy(d_hbm.at[pl.ds(wid * eg, eg)], idxd)
        bufm = (bufm0, bufm1)
        sr = (sr0, sr1)
        ss = (ss0, ss1)

        def issue_read(sl, g):
            pltpu.async_copy(m_hbm.at[pl.ds(wid * (eg * 128) + g * 128, 128)],
                             bufm[sl], sr[sl])

        def wait_read(sl):
            pltpu.make_async_copy(m_hbm.at[pl.ds(0, 128)], bufm[sl],
                                  sr[sl]).wait()

        def issue_scat(sl, g):
            pltpu.async_copy(bufm[sl], spm.at[idxd.at[g]], ss[sl], add=True)

        def wait_scat(sl, g):
            pltpu.make_async_copy(bufm[sl], spm.at[idxd.at[g]],
                                  ss[sl]).wait()

        issue_read(0, 0)

        @pl.loop(0, eg, step=2)
        def _(g):
            @pl.when(g >= 2)
            def _():
                wait_scat(1, g - 1)

            @pl.when(g + 1 < eg)
            def _():
                issue_read(1, g + 1)

            wait_read(0)
            issue_scat(0, g)
            wait_scat(0, g)

            @pl.when(g + 2 < eg)
            def _():
                issue_read(0, g + 2)

            @pl.when(g + 1 < eg)
            def _():
                wait_read(1)
                issue_scat(1, g + 1)

        wait_scat(1, eg - 1)
        plsc.subcore_barrier()
        pltpu.sync_copy(spm.at[pl.ds(sid * NROWS_PER_SUB, NROWS_PER_SUB)],
                        o_hbm.at[cid, pl.ds(sid * NROWS_PER_SUB, NROWS_PER_SUB)])

    return k(msgs, dst2d, zeros_tab)


def _dot(a, b):
    return jnp.dot(a, b, preferred_element_type=F32)


def _tc_pool_pos(pos_pad, batch_lanes):
    def body(pos_ref, bl_ref, pooled_ref, counts_ref):
        i = pl.program_id(0)
        brow = bl_ref[0][0:1, :]
        oht = (lax.broadcasted_iota(I32, (128, NBLK), 0) == brow).astype(F32)
        p = _dot(oht, pos_ref[...])
        c = _dot(oht, jnp.ones((NBLK, 128), F32))

        @pl.when(i == 0)
        def _():
            pooled_ref[...] = jnp.zeros_like(pooled_ref)
            counts_ref[...] = jnp.zeros_like(counts_ref)

        pooled_ref[...] += p
        counts_ref[...] += c

    return pl.pallas_call(
        body,
        grid=(NPAD // NBLK,),
        in_specs=[
            pl.BlockSpec((NBLK, 128), lambda i: (i, 0)),
            pl.BlockSpec((1, 8, NBLK), lambda i: (i, 0, 0)),
        ],
        out_specs=[
            pl.BlockSpec((128, 128), lambda i: (0, 0)),
            pl.BlockSpec((128, 128), lambda i: (0, 0)),
        ],
        out_shape=[jax.ShapeDtypeStruct((128, 128), F32),
                   jax.ShapeDtypeStruct((128, 128), F32)],
    )(pos_pad, batch_lanes)


def _tc_locmean(pos_pad, batchcol, pooled, countsm):
    def body(pos_ref, bc_ref, pooled_ref, counts_ref, out_ref):
        cent = pooled_ref[...] / jnp.maximum(counts_ref[...], 1.0)
        oh = (bc_ref[...] ==
              lax.broadcasted_iota(I32, (NBLK, 128), 1)).astype(F32)
        out_ref[...] = pos_ref[...] - _dot(oh, cent)

    return pl.pallas_call(
        body,
        grid=(NPAD // NBLK,),
        in_specs=[
            pl.BlockSpec((NBLK, 128), lambda i: (i, 0)),
            pl.BlockSpec((NBLK, 128), lambda i: (i, 0)),
            pl.BlockSpec((128, 128), lambda i: (0, 0)),
            pl.BlockSpec((128, 128), lambda i: (0, 0)),
        ],
        out_specs=pl.BlockSpec((NBLK, 128), lambda i: (i, 0)),
        out_shape=jax.ShapeDtypeStruct((NPAD, 128), F32),
    )(pos_pad, batchcol, pooled, countsm)


def _tc_embed(gpad, ntcol, wemb, wm1, bm1row, n_real):
    def body(g_ref, nt_ref, we_ref, wm1_ref, bm1_ref, x_ref, a_ref, b_ref):
        g = g_ref[...]
        e0 = _dot(g, we_ref[0])
        e1 = _dot(g, we_ref[1])
        e2 = _dot(g, we_ref[2])
        nt = nt_ref[...]
        x = jnp.where(nt == 0, e0, jnp.where(nt == 1, e1, e2))
        i = pl.program_id(0)
        row = lax.broadcasted_iota(I32, (NBLK, 128), 0) + i * NBLK
        x = jnp.where(row < n_real, x, 0.0)
        x_ref[...] = x
        a_ref[...] = _dot(x, wm1_ref[0:H, :]) + bm1_ref[0:1, :]
        b_ref[...] = _dot(x, wm1_ref[H:2 * H, :])

    return pl.pallas_call(
        body,
        grid=(NPAD // NBLK,),
        in_specs=[
            pl.BlockSpec((NBLK, 128), lambda i: (i, 0)),
            pl.BlockSpec((NBLK, 128), lambda i: (i, 0)),
            pl.BlockSpec((3, 128, 128), lambda i: (0, 0, 0)),
            pl.BlockSpec((2 * H, H), lambda i: (0, 0)),
            pl.BlockSpec((8, 128), lambda i: (0, 0)),
        ],
        out_specs=[pl.BlockSpec((NBLK, 128), lambda i: (i, 0))] * 3,
        out_shape=[jax.ShapeDtypeStruct((NPAD, 128), F32)] * 3,
    )(gpad, ntcol, wemb, wm1, bm1row)


def _tc_edge_mlp(grows, wm2, bm2row):
    def body(g_ref, w2_ref, b2_ref, out_ref):
        gsum = jnp.maximum(g_ref[...], 0.0)
        out_ref[...] = jnp.maximum(_dot(gsum, w2_ref[...]) + b2_ref[0:1, :],
                                   0.0)

    ne = grows.shape[0]
    return pl.pallas_call(
        body,
        grid=(ne // EBLK,),
        in_specs=[
            pl.BlockSpec((EBLK, H), lambda i: (i, 0)),
            pl.BlockSpec((H, H), lambda i: (0, 0)),
            pl.BlockSpec((8, 128), lambda i: (0, 0)),
        ],
        out_specs=pl.BlockSpec((EBLK, H), lambda i: (i, 0)),
        out_shape=jax.ShapeDtypeStruct((ne, H), F32),
    )(grows, wm2, bm2row)


def _tc_update(x, aggp0, aggp1, wu1, bu1row, wu2, bu2row, wm1n, bm1nrow,
               n_real):
    def body(x_ref, p_ref, q_ref, wu1_ref, bu1_ref, wu2_ref, bu2_ref,
             wm1_ref, bm1_ref, xo_ref, ao_ref, bo_ref):
        agg = p_ref[0] + p_ref[1] + q_ref[0] + q_ref[1]
        x_in = x_ref[...]
        h = jnp.maximum(_dot(x_in, wu1_ref[0:H, :]) +
                        _dot(agg, wu1_ref[H:2 * H, :]) + bu1_ref[0:1, :], 0.0)
        xn = _dot(h, wu2_ref[...]) + bu2_ref[0:1, :]
        i = pl.program_id(0)
        row = lax.broadcasted_iota(I32, (NBLK, 128), 0) + i * NBLK
        xn = jnp.where(row < n_real, xn, 0.0)
        xo_ref[...] = xn
        ao_ref[...] = _dot(xn, wm1_ref[0:H, :]) + bm1_ref[0:1, :]
        bo_ref[...] = _dot(xn, wm1_ref[H:2 * H, :])

    return pl.pallas_call(
        body,
        grid=(NPAD // NBLK,),
        in_specs=[
            pl.BlockSpec((NBLK, 128), lambda i: (i, 0)),
            pl.BlockSpec((2, NBLK, 128), lambda i: (0, i, 0)),
            pl.BlockSpec((2, NBLK, 128), lambda i: (0, i, 0)),
            pl.BlockSpec((2 * H, H), lambda i: (0, 0)),
            pl.BlockSpec((8, 128), lambda i: (0, 0)),
            pl.BlockSpec((H, H), lambda i: (0, 0)),
            pl.BlockSpec((8, 128), lambda i: (0, 0)),
            pl.BlockSpec((2 * H, H), lambda i: (0, 0)),
            pl.BlockSpec((8, 128), lambda i: (0, 0)),
        ],
        out_specs=[pl.BlockSpec((NBLK, 128), lambda i: (i, 0))] * 3,
        out_shape=[jax.ShapeDtypeStruct((NPAD, 128), F32)] * 3,
    )(x, aggp0, aggp1, wu1, bu1row, wu2, bu2row, wm1n, bm1nrow)


def _tc_update_final(x, aggp0, aggp1, wu1, bu1row, wu2, bu2row, batch_lanes,
                     n_real):
    def body(x_ref, p_ref, q_ref, wu1_ref, bu1_ref, wu2_ref, bu2_ref, bl_ref,
             pooled_ref):
        agg = p_ref[0] + p_ref[1] + q_ref[0] + q_ref[1]
        x_in = x_ref[...]
        h = jnp.maximum(_dot(x_in, wu1_ref[0:H, :]) +
                        _dot(agg, wu1_ref[H:2 * H, :]) + bu1_ref[0:1, :], 0.0)
        xn = _dot(h, wu2_ref[...]) + bu2_ref[0:1, :]
        i = pl.program_id(0)
        row = lax.broadcasted_iota(I32, (NBLK, 128), 0) + i * NBLK
        xn = jnp.where(row < n_real, xn, 0.0)
        brow = bl_ref[0][0:1, :]
        oht = (lax.broadcasted_iota(I32, (128, NBLK), 0) == brow).astype(F32)
        p = _dot(oht, xn)

        @pl.when(i == 0)
        def _():
            pooled_ref[...] = jnp.zeros_like(pooled_ref)

        pooled_ref[...] += p

    return pl.pallas_call(
        body,
        grid=(NPAD // NBLK,),
        in_specs=[
            pl.BlockSpec((NBLK, 128), lambda i: (i, 0)),
            pl.BlockSpec((2, NBLK, 128), lambda i: (0, i, 0)),
            pl.BlockSpec((2, NBLK, 128), lambda i: (0, i, 0)),
            pl.BlockSpec((2 * H, H), lambda i: (0, 0)),
            pl.BlockSpec((8, 128), lambda i: (0, 0)),
            pl.BlockSpec((H, H), lambda i: (0, 0)),
            pl.BlockSpec((8, 128), lambda i: (0, 0)),
            pl.BlockSpec((1, 8, NBLK), lambda i: (i, 0, 0)),
        ],
        out_specs=pl.BlockSpec((128, 128), lambda i: (0, 0)),
        out_shape=jax.ShapeDtypeStruct((128, 128), F32),
    )(x, aggp0, aggp1, wu1, bu1row, wu2, bu2row, batch_lanes)


def _tc_head(pooled, countsm, wp_pad, bprow, ypad, n_graphs):
    def body(pooled_ref, counts_ref, wp_ref, bp_ref, y_ref,
             loss_ref, acc_ref, mloss_ref):
        pm = pooled_ref[...] / jnp.maximum(counts_ref[...], 1.0)
        out = _dot(pm, wp_ref[...]) + bp_ref[0:1, :]
        lane = lax.broadcasted_iota(I32, (128, 128), 1)
        row = lax.broadcasted_iota(I32, (128, 128), 0)
        z = jnp.where(lane < 2, out, -1e30)
        m = jnp.max(z, axis=1, keepdims=True)
        s = jnp.sum(jnp.where(lane < 2, jnp.exp(z - m), 0.0), axis=1,
                    keepdims=True)
        lse = m + jnp.log(s)
        y = y_ref[...]
        lossv = -jnp.sum(jnp.where(lane < 2, y * (out - lse), 0.0), axis=1,
                         keepdims=True)
        o0 = out[:, 0:1]
        o1 = out[:, 1:2]
        y0 = y[:, 0:1]
        y1 = y[:, 1:2]
        accv = ((o1 > o0) == (y1 > y0)).astype(F32)
        loss_ref[...] = jnp.broadcast_to(lossv, (128, 128))
        acc_ref[...] = jnp.broadcast_to(accv, (128, 128))
        total = jnp.sum(jnp.where(row[:, 0:1] < n_graphs, lossv, 0.0))
        mloss_ref[...] = jnp.full((8, 128), total / n_graphs, F32)

    return pl.pallas_call(
        body,
        grid=(1,),
        in_specs=[pl.BlockSpec((128, 128), lambda i: (0, 0))] * 5,
        out_specs=[
            pl.BlockSpec((128, 128), lambda i: (0, 0)),
            pl.BlockSpec((128, 128), lambda i: (0, 0)),
            pl.BlockSpec((8, 128), lambda i: (0, 0)),
        ],
        out_shape=[jax.ShapeDtypeStruct((128, 128), F32),
                   jax.ShapeDtypeStruct((128, 128), F32),
                   jax.ShapeDtypeStruct((8, 128), F32)],
    )(pooled, countsm, wp_pad, bprow, ypad)


def _rowpad(b):
    return jnp.broadcast_to(b[None, :], (8, 128)).astype(F32)


def kernel(pos, y, params, node_types, x_ind, batch, edge_index):
    n = pos.shape[0]
    e = edge_index.shape[1]
    nb = y.shape[0]

    pos_pad = jnp.zeros((NPAD, 128), F32).at[:n, :3].set(pos.astype(F32))
    batch_pad = jnp.concatenate(
        [batch.astype(I32), jnp.full((NPAD - n,), nb, I32)])
    batchcol = jnp.broadcast_to(batch_pad[:, None], (NPAD, 128))
    batch_lanes = jnp.broadcast_to(
        batch_pad.reshape(NPAD // NBLK, 1, NBLK), (NPAD // NBLK, 8, NBLK))
    nt_pad = jnp.concatenate(
        [node_types.astype(I32), jnp.zeros((NPAD - n,), I32)])
    ntcol = jnp.broadcast_to(nt_pad[:, None], (NPAD, 128))

    src = edge_index[0].astype(I32)
    dst = edge_index[1].astype(I32)
    epad_ids = jnp.arange(EPAD - e, dtype=I32)
    src2d = jnp.concatenate([src, epad_ids % n]).reshape(-1, 128)
    dst2d = jnp.concatenate([dst, n + epad_ids % (NPAD - n)]).reshape(-1, 128)

    xif = x_ind.reshape(-1).astype(I32)
    xif2d = jnp.concatenate(
        [xif, jnp.arange(XPAD - 3 * n, dtype=I32) % n]).reshape(-1, 128)

    zeros_tab = jnp.zeros((NPAD, H), F32)

    wemb = jnp.zeros((3, 128, 128), F32)
    for i in range(3):
        wi = params['emb'][i]['W']
        for j in range(i + 1):
            wemb = wemb.at[i, 16 * j:16 * j + 3, :].set(wi[3 * j:3 * j + 3, :])
        wemb = wemb.at[i, 127, :].set(params['emb'][i]['b'])

    wp_pad = jnp.zeros((128, 128), F32).at[:, :2].set(params['proj']['W'])
    bprow = jnp.zeros((8, 128), F32).at[0, :2].set(params['proj']['b'])
    bprow = jnp.broadcast_to(bprow[0:1, :], (8, 128))
    ypad = jnp.zeros((128, 128), F32).at[:nb, :2].set(y.astype(F32))

    layers = params['layers']

    pooled_pos, countsm = _tc_pool_pos(pos_pad, batch_lanes)
    locmean = _tc_locmean(pos_pad, batchcol, pooled_pos, countsm)

    g = _sc_gather16(locmean, xif2d)
    g48 = g[:3 * n].reshape(n, 3, 128)[:, :, :16].reshape(n, 48)
    gpad = jnp.zeros((NPAD, 128), F32).at[:n, :48].set(g48)
    gpad = gpad.at[:, 127].set(1.0)

    lp = layers[0]
    x, ta, tb = _tc_embed(gpad, ntcol, wemb, lp['Wm1'], _rowpad(lp['bm1']), n)

    hg = src2d.shape[0] // 2
    egh = hg // NWORK
    s2d = (src2d[:hg], src2d[hg:])
    d2d = (dst2d[:hg], dst2d[hg:])
    for l in range(3):
        lp = layers[l]
        bm2r = _rowpad(lp['bm2'])
        g0 = _sc_edge_gather(ta, tb, s2d[0], d2d[0], egh)
        g1 = _sc_edge_gather(ta, tb, s2d[1], d2d[1], egh)
        m0 = _tc_edge_mlp(g0, lp['Wm2'], bm2r)
        m1 = _tc_edge_mlp(g1, lp['Wm2'], bm2r)
        aggp0 = _sc_scatter_add(m0, d2d[0], zeros_tab, egh)
        aggp1 = _sc_scatter_add(m1, d2d[1], zeros_tab, egh)
        if l < 2:
            lpn = layers[l + 1]
            x, ta, tb = _tc_update(
                x, aggp0, aggp1, lp['Wu1'], _rowpad(lp['bu1']), lp['Wu2'],
                _rowpad(lp['bu2']), lpn['Wm1'], _rowpad(lpn['bm1']), n)
        else:
            pooled_x = _tc_update_final(
                x, aggp0, aggp1, lp['Wu1'], _rowpad(lp['bu1']), lp['Wu2'],
                _rowpad(lp['bu2']), batch_lanes, n)

    loss128, acc128, mloss = _tc_head(pooled_x, countsm, wp_pad, bprow, ypad,
                                      nb)

    loss = loss128[:nb, 0]
    acc = acc128[:nb, 0]
    backprop_loss = mloss[0, 0].reshape(())
    return backprop_loss, loss, acc


# speedup vs baseline: 1.0772x; 1.0221x over previous
import functools

import jax
import jax.numpy as jnp
from jax import lax
from jax.experimental import pallas as pl
from jax.experimental.pallas import tpu as pltpu
from jax.experimental.pallas import tpu_sc as plsc

F32 = jnp.float32
BF16 = jnp.bfloat16
I32 = jnp.int32

H = 128
NPAD = 10240
EPAD = 327680
XPAD = 32768
NBLK = 2048
EBLK = 4096
NWORK = 32
EG = 80
NROWS_PER_SUB = NPAD // 16

@functools.cache
def _mesh():
    return plsc.VectorSubcoreMesh(core_axis_name="c", subcore_axis_name="s",
                                  num_cores=2, num_subcores=16)


def _sc_gather16(table, idx2d):
    @functools.partial(
        pl.kernel,
        out_type=jax.ShapeDtypeStruct((XPAD, 128), F32),
        mesh=_mesh(),
        scratch_types=[
            pltpu.VMEM((8, 128), I32),
            pltpu.VMEM((128, 128), F32),
            pltpu.SemaphoreType.DMA,
        ],
    )
    def k(tab_hbm, idx_hbm, out_hbm, idx_v, buf, sem):
        wid = lax.axis_index("s") * 2 + lax.axis_index("c")
        pltpu.sync_copy(idx_hbm.at[pl.ds(wid * 8, 8)], idx_v)

        @pl.loop(0, 8)
        def _(g):
            pltpu.async_copy(tab_hbm.at[idx_v.at[g]], buf, sem).wait()
            pltpu.sync_copy(buf, out_hbm.at[pl.ds(wid * 1024 + g * 128, 128)])

    return k(table, idx2d)


def _sc_edge_gather(tab_a, tab_b, src2d, dst2d, eg):
    ne = NWORK * eg * 128
    @functools.partial(
        pl.kernel,
        out_type=jax.ShapeDtypeStruct((ne, H), F32),
        mesh=_mesh(),
        scratch_types=[
            pltpu.VMEM((eg, 128), I32),
            pltpu.VMEM((eg, 128), I32),
            pltpu.VMEM((128, H), F32),
            pltpu.VMEM((128, H), F32),
            pltpu.VMEM_SHARED((NPAD, H), F32),
        ] + [pltpu.SemaphoreType.DMA] * 4,
    )
    def k(a_hbm, b_hbm, s_hbm, d_hbm, o_hbm,
          idxa, idxb, buf0, buf1, spa,
          sg0, sg1, sw0, sw1):
        sid = lax.axis_index("s")
        wid = sid * 2 + lax.axis_index("c")
        pltpu.sync_copy(a_hbm.at[pl.ds(sid * NROWS_PER_SUB, NROWS_PER_SUB)],
                        spa.at[pl.ds(sid * NROWS_PER_SUB, NROWS_PER_SUB)])
        pltpu.sync_copy(s_hbm.at[pl.ds(wid * eg, eg)], idxa)
        pltpu.sync_copy(d_hbm.at[pl.ds(wid * eg, eg)], idxb)
        plsc.subcore_barrier()
        bufs = (buf0, buf1)
        sg = (sg0, sg1)
        sw = (sw0, sw1)

        def issue_ga(sl, g):
            pltpu.async_copy(spa.at[idxa.at[g]], bufs[sl], sg[sl])

        def wait_ga(sl, g):
            pltpu.make_async_copy(spa.at[idxa.at[g]], bufs[sl],
                                  sg[sl]).wait()

        def issue_gb(sl, g):
            pltpu.async_copy(b_hbm.at[idxb.at[g]], bufs[sl], sg[sl],
                             add=True)

        def wait_gb(sl, g):
            pltpu.make_async_copy(b_hbm.at[idxb.at[g]], bufs[sl],
                                  sg[sl]).wait()

        def issue_w(sl, g):
            row0 = wid * (eg * 128) + g * 128
            pltpu.async_copy(bufs[sl], o_hbm.at[pl.ds(row0, 128)], sw[sl])

        def wait_w(sl):
            pltpu.make_async_copy(bufs[sl], o_hbm.at[pl.ds(0, 128)],
                                  sw[sl]).wait()

        issue_ga(0, 0)

        @pl.loop(0, eg, step=2)
        def _(g):
            @pl.when(g >= 2)
            def _():
                wait_w(1)

            @pl.when(g + 1 < eg)
            def _():
                issue_ga(1, g + 1)

            wait_ga(0, g)
            issue_gb(0, g)
            wait_gb(0, g)
            issue_w(0, g)
            wait_w(0)

            @pl.when(g + 2 < eg)
            def _():
                issue_ga(0, g + 2)

            @pl.when(g + 1 < eg)
            def _():
                wait_ga(1, g + 1)
                issue_gb(1, g + 1)
                wait_gb(1, g + 1)
                issue_w(1, g + 1)

        wait_w(1)

    return k(tab_a, tab_b, src2d, dst2d)


def _sc_scatter_add(msgs, dst2d, zeros_tab, eg):
    @functools.partial(
        pl.kernel,
        out_type=jax.ShapeDtypeStruct((2, NPAD, H), F32),
        mesh=_mesh(),
        scratch_types=[
            pltpu.VMEM((eg, 128), I32),
            pltpu.VMEM((128, H), F32),
            pltpu.VMEM((128, H), F32),
            pltpu.VMEM_SHARED((NPAD, H), F32),
        ] + [pltpu.SemaphoreType.DMA] * 4,
    )
    def k(m_hbm, d_hbm, z_hbm, o_hbm, idxd, bufm0, bufm1, spm,
          sr0, sr1, ss0, ss1):
        cid = lax.axis_index("c")
        sid = lax.axis_index("s")
        wid = sid * 2 + cid
        pltpu.sync_copy(z_hbm.at[pl.ds(sid * NROWS_PER_SUB, NROWS_PER_SUB)],
                        spm.at[pl.ds(sid * NROWS_PER_SUB, NROWS_PER_SUB)])
        plsc.subcore_barrier()
        pltpu.sync_copy(d_hbm.at[pl.ds(wid * eg, eg)], idxd)
        bufm = (bufm0, bufm1)
        sr = (sr0, sr1)
        ss = (ss0, ss1)

        def issue_read(sl, g):
            pltpu.async_copy(m_hbm.at[pl.ds(wid * (eg * 128) + g * 128, 128)],
                             bufm[sl], sr[sl])

        def wait_read(sl):
            pltpu.make_async_copy(m_hbm.at[pl.ds(0, 128)], bufm[sl],
                                  sr[sl]).wait()

        def issue_scat(sl, g):
            pltpu.async_copy(bufm[sl], spm.at[idxd.at[g]], ss[sl], add=True)

        def wait_scat(sl, g):
            pltpu.make_async_copy(bufm[sl], spm.at[idxd.at[g]],
                                  ss[sl]).wait()

        issue_read(0, 0)

        @pl.loop(0, eg, step=2)
        def _(g):
            @pl.when(g >= 2)
            def _():
                wait_scat(1, g - 1)

            @pl.when(g + 1 < eg)
            def _():
                issue_read(1, g + 1)

            wait_read(0)
            issue_scat(0, g)
            wait_scat(0, g)

            @pl.when(g + 2 < eg)
            def _():
                issue_read(0, g + 2)

            @pl.when(g + 1 < eg)
            def _():
                wait_read(1)
                issue_scat(1, g + 1)

        wait_scat(1, eg - 1)
        plsc.subcore_barrier()
        pltpu.sync_copy(spm.at[pl.ds(sid * NROWS_PER_SUB, NROWS_PER_SUB)],
                        o_hbm.at[cid, pl.ds(sid * NROWS_PER_SUB, NROWS_PER_SUB)])

    return k(msgs, dst2d, zeros_tab)


def _dot(a, b):
    return jnp.dot(a, b, preferred_element_type=F32)


def _tc_pool_pos(pos_pad, batch_lanes):
    def body(pos_ref, bl_ref, pooled_ref, counts_ref):
        i = pl.program_id(0)
        brow = bl_ref[0][0:1, :]
        oht = (lax.broadcasted_iota(I32, (128, NBLK), 0) == brow).astype(F32)
        p = _dot(oht, pos_ref[...])
        c = _dot(oht, jnp.ones((NBLK, 128), F32))

        @pl.when(i == 0)
        def _():
            pooled_ref[...] = jnp.zeros_like(pooled_ref)
            counts_ref[...] = jnp.zeros_like(counts_ref)

        pooled_ref[...] += p
        counts_ref[...] += c

    return pl.pallas_call(
        body,
        grid=(NPAD // NBLK,),
        in_specs=[
            pl.BlockSpec((NBLK, 128), lambda i: (i, 0)),
            pl.BlockSpec((1, 8, NBLK), lambda i: (i, 0, 0)),
        ],
        out_specs=[
            pl.BlockSpec((128, 128), lambda i: (0, 0)),
            pl.BlockSpec((128, 128), lambda i: (0, 0)),
        ],
        out_shape=[jax.ShapeDtypeStruct((128, 128), F32),
                   jax.ShapeDtypeStruct((128, 128), F32)],
    )(pos_pad, batch_lanes)


def _tc_locmean(pos_pad, batchcol, pooled, countsm):
    def body(pos_ref, bc_ref, pooled_ref, counts_ref, out_ref):
        cent = pooled_ref[...] / jnp.maximum(counts_ref[...], 1.0)
        oh = (bc_ref[...] ==
              lax.broadcasted_iota(I32, (NBLK, 128), 1)).astype(F32)
        out_ref[...] = pos_ref[...] - _dot(oh, cent)

    return pl.pallas_call(
        body,
        grid=(NPAD // NBLK,),
        in_specs=[
            pl.BlockSpec((NBLK, 128), lambda i: (i, 0)),
            pl.BlockSpec((NBLK, 128), lambda i: (i, 0)),
            pl.BlockSpec((128, 128), lambda i: (0, 0)),
            pl.BlockSpec((128, 128), lambda i: (0, 0)),
        ],
        out_specs=pl.BlockSpec((NBLK, 128), lambda i: (i, 0)),
        out_shape=jax.ShapeDtypeStruct((NPAD, 128), F32),
    )(pos_pad, batchcol, pooled, countsm)


def _tc_embed(gpad, ntcol, wemb, wm1, bm1row, n_real):
    def body(g_ref, nt_ref, we_ref, wm1_ref, bm1_ref, x_ref, a_ref, b_ref):
        g = g_ref[...]
        e0 = _dot(g, we_ref[0])
        e1 = _dot(g, we_ref[1])
        e2 = _dot(g, we_ref[2])
        nt = nt_ref[...]
        x = jnp.where(nt == 0, e0, jnp.where(nt == 1, e1, e2))
        i = pl.program_id(0)
        row = lax.broadcasted_iota(I32, (NBLK, 128), 0) + i * NBLK
        x = jnp.where(row < n_real, x, 0.0)
        x_ref[...] = x
        a_ref[...] = _dot(x, wm1_ref[0:H, :]) + bm1_ref[0:1, :]
        b_ref[...] = _dot(x, wm1_ref[H:2 * H, :])

    return pl.pallas_call(
        body,
        grid=(NPAD // NBLK,),
        in_specs=[
            pl.BlockSpec((NBLK, 128), lambda i: (i, 0)),
            pl.BlockSpec((NBLK, 128), lambda i: (i, 0)),
            pl.BlockSpec((3, 128, 128), lambda i: (0, 0, 0)),
            pl.BlockSpec((2 * H, H), lambda i: (0, 0)),
            pl.BlockSpec((8, 128), lambda i: (0, 0)),
        ],
        out_specs=[pl.BlockSpec((NBLK, 128), lambda i: (i, 0))] * 3,
        out_shape=[jax.ShapeDtypeStruct((NPAD, 128), F32)] * 3,
    )(gpad, ntcol, wemb, wm1, bm1row)


def _tc_edge_mlp(grows, wm2, bm2row):
    def body(g_ref, w2_ref, b2_ref, out_ref):
        gsum = jnp.maximum(g_ref[...], 0.0)
        out_ref[...] = jnp.maximum(_dot(gsum, w2_ref[...]) + b2_ref[0:1, :],
                                   0.0)

    ne = grows.shape[0]
    return pl.pallas_call(
        body,
        grid=(ne // EBLK,),
        in_specs=[
            pl.BlockSpec((EBLK, H), lambda i: (i, 0)),
            pl.BlockSpec((H, H), lambda i: (0, 0)),
            pl.BlockSpec((8, 128), lambda i: (0, 0)),
        ],
        out_specs=pl.BlockSpec((EBLK, H), lambda i: (i, 0)),
        out_shape=jax.ShapeDtypeStruct((ne, H), F32),
    )(grows, wm2, bm2row)


def _tc_update(x, aggp0, aggp1, wu1, bu1row, wu2, bu2row, wm1n, bm1nrow,
               n_real):
    def body(x_ref, p_ref, q_ref, wu1_ref, bu1_ref, wu2_ref, bu2_ref,
             wm1_ref, bm1_ref, xo_ref, ao_ref, bo_ref):
        agg = p_ref[0] + p_ref[1] + q_ref[0] + q_ref[1]
        x_in = x_ref[...]
        h = jnp.maximum(_dot(x_in, wu1_ref[0:H, :]) +
                        _dot(agg, wu1_ref[H:2 * H, :]) + bu1_ref[0:1, :], 0.0)
        xn = _dot(h, wu2_ref[...]) + bu2_ref[0:1, :]
        i = pl.program_id(0)
        row = lax.broadcasted_iota(I32, (NBLK, 128), 0) + i * NBLK
        xn = jnp.where(row < n_real, xn, 0.0)
        xo_ref[...] = xn
        ao_ref[...] = _dot(xn, wm1_ref[0:H, :]) + bm1_ref[0:1, :]
        bo_ref[...] = _dot(xn, wm1_ref[H:2 * H, :])

    return pl.pallas_call(
        body,
        grid=(NPAD // NBLK,),
        in_specs=[
            pl.BlockSpec((NBLK, 128), lambda i: (i, 0)),
            pl.BlockSpec((2, NBLK, 128), lambda i: (0, i, 0)),
            pl.BlockSpec((2, NBLK, 128), lambda i: (0, i, 0)),
            pl.BlockSpec((2 * H, H), lambda i: (0, 0)),
            pl.BlockSpec((8, 128), lambda i: (0, 0)),
            pl.BlockSpec((H, H), lambda i: (0, 0)),
            pl.BlockSpec((8, 128), lambda i: (0, 0)),
            pl.BlockSpec((2 * H, H), lambda i: (0, 0)),
            pl.BlockSpec((8, 128), lambda i: (0, 0)),
        ],
        out_specs=[pl.BlockSpec((NBLK, 128), lambda i: (i, 0))] * 3,
        out_shape=[jax.ShapeDtypeStruct((NPAD, 128), F32)] * 3,
    )(x, aggp0, aggp1, wu1, bu1row, wu2, bu2row, wm1n, bm1nrow)


def _tc_update_final(x, aggp0, aggp1, wu1, bu1row, wu2, bu2row, batch_lanes,
                     n_real):
    def body(x_ref, p_ref, q_ref, wu1_ref, bu1_ref, wu2_ref, bu2_ref, bl_ref,
             pooled_ref):
        agg = p_ref[0] + p_ref[1] + q_ref[0] + q_ref[1]
        x_in = x_ref[...]
        h = jnp.maximum(_dot(x_in, wu1_ref[0:H, :]) +
                        _dot(agg, wu1_ref[H:2 * H, :]) + bu1_ref[0:1, :], 0.0)
        xn = _dot(h, wu2_ref[...]) + bu2_ref[0:1, :]
        i = pl.program_id(0)
        row = lax.broadcasted_iota(I32, (NBLK, 128), 0) + i * NBLK
        xn = jnp.where(row < n_real, xn, 0.0)
        brow = bl_ref[0][0:1, :]
        oht = (lax.broadcasted_iota(I32, (128, NBLK), 0) == brow).astype(F32)
        p = _dot(oht, xn)

        @pl.when(i == 0)
        def _():
            pooled_ref[...] = jnp.zeros_like(pooled_ref)

        pooled_ref[...] += p

    return pl.pallas_call(
        body,
        grid=(NPAD // NBLK,),
        in_specs=[
            pl.BlockSpec((NBLK, 128), lambda i: (i, 0)),
            pl.BlockSpec((2, NBLK, 128), lambda i: (0, i, 0)),
            pl.BlockSpec((2, NBLK, 128), lambda i: (0, i, 0)),
            pl.BlockSpec((2 * H, H), lambda i: (0, 0)),
            pl.BlockSpec((8, 128), lambda i: (0, 0)),
            pl.BlockSpec((H, H), lambda i: (0, 0)),
            pl.BlockSpec((8, 128), lambda i: (0, 0)),
            pl.BlockSpec((1, 8, NBLK), lambda i: (i, 0, 0)),
        ],
        out_specs=pl.BlockSpec((128, 128), lambda i: (0, 0)),
        out_shape=jax.ShapeDtypeStruct((128, 128), F32),
    )(x, aggp0, aggp1, wu1, bu1row, wu2, bu2row, batch_lanes)


def _tc_head(pooled, countsm, wp_pad, bprow, ypad, n_graphs):
    def body(pooled_ref, counts_ref, wp_ref, bp_ref, y_ref,
             loss_ref, acc_ref, mloss_ref):
        pm = pooled_ref[...] / jnp.maximum(counts_ref[...], 1.0)
        out = _dot(pm, wp_ref[...]) + bp_ref[0:1, :]
        lane = lax.broadcasted_iota(I32, (128, 128), 1)
        row = lax.broadcasted_iota(I32, (128, 128), 0)
        z = jnp.where(lane < 2, out, -1e30)
        m = jnp.max(z, axis=1, keepdims=True)
        s = jnp.sum(jnp.where(lane < 2, jnp.exp(z - m), 0.0), axis=1,
                    keepdims=True)
        lse = m + jnp.log(s)
        y = y_ref[...]
        lossv = -jnp.sum(jnp.where(lane < 2, y * (out - lse), 0.0), axis=1,
                         keepdims=True)
        o0 = out[:, 0:1]
        o1 = out[:, 1:2]
        y0 = y[:, 0:1]
        y1 = y[:, 1:2]
        accv = ((o1 > o0) == (y1 > y0)).astype(F32)
        loss_ref[...] = jnp.broadcast_to(lossv, (128, 128))
        acc_ref[...] = jnp.broadcast_to(accv, (128, 128))
        total = jnp.sum(jnp.where(row[:, 0:1] < n_graphs, lossv, 0.0))
        mloss_ref[...] = jnp.full((8, 128), total / n_graphs, F32)

    return pl.pallas_call(
        body,
        grid=(1,),
        in_specs=[pl.BlockSpec((128, 128), lambda i: (0, 0))] * 5,
        out_specs=[
            pl.BlockSpec((128, 128), lambda i: (0, 0)),
            pl.BlockSpec((128, 128), lambda i: (0, 0)),
            pl.BlockSpec((8, 128), lambda i: (0, 0)),
        ],
        out_shape=[jax.ShapeDtypeStruct((128, 128), F32),
                   jax.ShapeDtypeStruct((128, 128), F32),
                   jax.ShapeDtypeStruct((8, 128), F32)],
    )(pooled, countsm, wp_pad, bprow, ypad)


def _rowpad(b):
    return jnp.broadcast_to(b[None, :], (8, 128)).astype(F32)


def kernel(pos, y, params, node_types, x_ind, batch, edge_index):
    n = pos.shape[0]
    e = edge_index.shape[1]
    nb = y.shape[0]

    pos_pad = jnp.zeros((NPAD, 128), F32).at[:n, :3].set(pos.astype(F32))
    batch_pad = jnp.concatenate(
        [batch.astype(I32), jnp.full((NPAD - n,), nb, I32)])
    batchcol = jnp.broadcast_to(batch_pad[:, None], (NPAD, 128))
    batch_lanes = jnp.broadcast_to(
        batch_pad.reshape(NPAD // NBLK, 1, NBLK), (NPAD // NBLK, 8, NBLK))
    nt_pad = jnp.concatenate(
        [node_types.astype(I32), jnp.zeros((NPAD - n,), I32)])
    ntcol = jnp.broadcast_to(nt_pad[:, None], (NPAD, 128))

    src = edge_index[0].astype(I32)
    dst = edge_index[1].astype(I32)
    epad_ids = jnp.arange(EPAD - e, dtype=I32)
    src2d = jnp.concatenate([src, epad_ids % n]).reshape(-1, 128)
    dst2d = jnp.concatenate([dst, n + epad_ids % (NPAD - n)]).reshape(-1, 128)

    xif = x_ind.reshape(-1).astype(I32)
    xif2d = jnp.concatenate(
        [xif, jnp.arange(XPAD - 3 * n, dtype=I32) % n]).reshape(-1, 128)

    zeros_tab = jnp.zeros((NPAD, H), F32)

    wemb = jnp.zeros((3, 128, 128), F32)
    for i in range(3):
        wi = params['emb'][i]['W']
        for j in range(i + 1):
            wemb = wemb.at[i, 16 * j:16 * j + 3, :].set(wi[3 * j:3 * j + 3, :])
        wemb = wemb.at[i, 127, :].set(params['emb'][i]['b'])

    wp_pad = jnp.zeros((128, 128), F32).at[:, :2].set(params['proj']['W'])
    bprow = jnp.zeros((8, 128), F32).at[0, :2].set(params['proj']['b'])
    bprow = jnp.broadcast_to(bprow[0:1, :], (8, 128))
    ypad = jnp.zeros((128, 128), F32).at[:nb, :2].set(y.astype(F32))

    layers = params['layers']

    pooled_pos, countsm = _tc_pool_pos(pos_pad, batch_lanes)
    locmean = _tc_locmean(pos_pad, batchcol, pooled_pos, countsm)

    g = _sc_gather16(locmean, xif2d)
    g48 = g[:3 * n].reshape(n, 3, 128)[:, :, :16].reshape(n, 48)
    gpad = jnp.zeros((NPAD, 128), F32).at[:n, :48].set(g48)
    gpad = gpad.at[:, 127].set(1.0)

    lp = layers[0]
    x, ta, tb = _tc_embed(gpad, ntcol, wemb, lp['Wm1'], _rowpad(lp['bm1']), n)

    hg = src2d.shape[0] // 2
    egh = hg // NWORK
    s2d = (src2d[:hg], src2d[hg:])
    d2d = (dst2d[:hg], dst2d[hg:])
    for l in range(3):
        lp = layers[l]
        bm2r = _rowpad(lp['bm2'])
        g0 = _sc_edge_gather(ta, tb, s2d[0], d2d[0], egh)
        g1 = _sc_edge_gather(ta, tb, s2d[1], d2d[1], egh)
        m0 = _tc_edge_mlp(g0, lp['Wm2'], bm2r)
        m1 = _tc_edge_mlp(g1, lp['Wm2'], bm2r)
        aggp0 = _sc_scatter_add(m0, d2d[0], zeros_tab, egh)
        aggp1 = _sc_scatter_add(m1, d2d[1], zeros_tab, egh)
        if l < 2:
            lpn = layers[l + 1]
            x, ta, tb = _tc_update(
                x, aggp0, aggp1, lp['Wu1'], _rowpad(lp['bu1']), lp['Wu2'],
                _rowpad(lp['bu2']), lpn['Wm1'], _rowpad(lpn['bm1']), n)
        else:
            pooled_x = _tc_update_final(
                x, aggp0, aggp1, lp['Wu1'], _rowpad(lp['bu1']), lp['Wu2'],
                _rowpad(lp['bu2']), batch_lanes, n)

    loss128, acc128, mloss = _tc_head(pooled_x, countsm, wp_pad, bprow, ypad,
                                      nb)

    loss = loss128[:nb, 0]
    acc = acc128[:nb, 0]
    backprop_loss = mloss[0, 0].reshape(())
    return backprop_loss, loss, acc


# speedup vs baseline: 1.1279x; 1.0471x over previous
import functools

import jax
import jax.numpy as jnp
from jax import lax
from jax.experimental import pallas as pl
from jax.experimental.pallas import tpu as pltpu
from jax.experimental.pallas import tpu_sc as plsc

F32 = jnp.float32
BF16 = jnp.bfloat16
I32 = jnp.int32

H = 128
NPAD = 10240
EPAD = 327680
XPAD = 32768
NBLK = 2048
EBLK = 4096
NWORK = 32
EG = 80
NROWS_PER_SUB = NPAD // 16

@functools.cache
def _mesh():
    return plsc.VectorSubcoreMesh(core_axis_name="c", subcore_axis_name="s",
                                  num_cores=2, num_subcores=16)


def _sc_gather16(table, idx2d):
    @functools.partial(
        pl.kernel,
        out_type=jax.ShapeDtypeStruct((XPAD, 128), F32),
        mesh=_mesh(),
        scratch_types=[
            pltpu.VMEM((8, 128), I32),
            pltpu.VMEM((128, 128), F32),
            pltpu.SemaphoreType.DMA,
        ],
    )
    def k(tab_hbm, idx_hbm, out_hbm, idx_v, buf, sem):
        wid = lax.axis_index("s") * 2 + lax.axis_index("c")
        pltpu.sync_copy(idx_hbm.at[pl.ds(wid * 8, 8)], idx_v)

        @pl.loop(0, 8)
        def _(g):
            pltpu.async_copy(tab_hbm.at[idx_v.at[g]], buf, sem).wait()
            pltpu.sync_copy(buf, out_hbm.at[pl.ds(wid * 1024 + g * 128, 128)])

    return k(table, idx2d)


def _sc_edge_gather(tab_a, tab_b, src2d, dst2d, eg):
    ne = NWORK * eg * 128
    @functools.partial(
        pl.kernel,
        out_type=jax.ShapeDtypeStruct((ne, H), F32),
        mesh=_mesh(),
        scratch_types=[
            pltpu.VMEM((eg, 128), I32),
            pltpu.VMEM((eg, 128), I32),
            pltpu.VMEM((128, H), F32),
            pltpu.VMEM((128, H), F32),
            pltpu.VMEM_SHARED((NPAD, H), F32),
        ] + [pltpu.SemaphoreType.DMA] * 4,
    )
    def k(a_hbm, b_hbm, s_hbm, d_hbm, o_hbm,
          idxa, idxb, buf0, buf1, spa,
          sg0, sg1, sw0, sw1):
        sid = lax.axis_index("s")
        wid = sid * 2 + lax.axis_index("c")
        pltpu.sync_copy(a_hbm.at[pl.ds(sid * NROWS_PER_SUB, NROWS_PER_SUB)],
                        spa.at[pl.ds(sid * NROWS_PER_SUB, NROWS_PER_SUB)])
        pltpu.sync_copy(s_hbm.at[pl.ds(wid * eg, eg)], idxa)
        pltpu.sync_copy(d_hbm.at[pl.ds(wid * eg, eg)], idxb)
        plsc.subcore_barrier()
        bufs = (buf0, buf1)
        sg = (sg0, sg1)
        sw = (sw0, sw1)

        def issue_ga(sl, g):
            pltpu.async_copy(spa.at[idxa.at[g]], bufs[sl], sg[sl])

        def wait_ga(sl, g):
            pltpu.make_async_copy(spa.at[idxa.at[g]], bufs[sl],
                                  sg[sl]).wait()

        def issue_gb(sl, g):
            pltpu.async_copy(b_hbm.at[idxb.at[g]], bufs[sl], sg[sl],
                             add=True)

        def wait_gb(sl, g):
            pltpu.make_async_copy(b_hbm.at[idxb.at[g]], bufs[sl],
                                  sg[sl]).wait()

        def issue_w(sl, g):
            row0 = wid * (eg * 128) + g * 128
            pltpu.async_copy(bufs[sl], o_hbm.at[pl.ds(row0, 128)], sw[sl])

        def wait_w(sl):
            pltpu.make_async_copy(bufs[sl], o_hbm.at[pl.ds(0, 128)],
                                  sw[sl]).wait()

        issue_ga(0, 0)

        @pl.loop(0, eg, step=2)
        def _(g):
            @pl.when(g >= 2)
            def _():
                wait_w(1)

            @pl.when(g + 1 < eg)
            def _():
                issue_ga(1, g + 1)

            wait_ga(0, g)
            issue_gb(0, g)
            wait_gb(0, g)
            issue_w(0, g)
            wait_w(0)

            @pl.when(g + 2 < eg)
            def _():
                issue_ga(0, g + 2)

            @pl.when(g + 1 < eg)
            def _():
                wait_ga(1, g + 1)
                issue_gb(1, g + 1)
                wait_gb(1, g + 1)
                issue_w(1, g + 1)

        wait_w(1)

    return k(tab_a, tab_b, src2d, dst2d)


def _sc_scatter_add(msgs, dst2d, zeros_tab, eg):
    @functools.partial(
        pl.kernel,
        out_type=jax.ShapeDtypeStruct((2, NPAD, H), F32),
        mesh=_mesh(),
        scratch_types=[
            pltpu.VMEM((eg, 128), I32),
            pltpu.VMEM((128, H), F32),
            pltpu.VMEM((128, H), F32),
            pltpu.VMEM_SHARED((NPAD, H), F32),
        ] + [pltpu.SemaphoreType.DMA] * 4,
    )
    def k(m_hbm, d_hbm, z_hbm, o_hbm, idxd, bufm0, bufm1, spm,
          sr0, sr1, ss0, ss1):
        cid = lax.axis_index("c")
        sid = lax.axis_index("s")
        wid = sid * 2 + cid
        pltpu.sync_copy(z_hbm.at[pl.ds(sid * NROWS_PER_SUB, NROWS_PER_SUB)],
                        spm.at[pl.ds(sid * NROWS_PER_SUB, NROWS_PER_SUB)])
        plsc.subcore_barrier()
        pltpu.sync_copy(d_hbm.at[pl.ds(wid * eg, eg)], idxd)
        bufm = (bufm0, bufm1)
        sr = (sr0, sr1)
        ss = (ss0, ss1)

        def issue_read(sl, g):
            pltpu.async_copy(m_hbm.at[pl.ds(wid * (eg * 128) + g * 128, 128)],
                             bufm[sl], sr[sl])

        def wait_read(sl):
            pltpu.make_async_copy(m_hbm.at[pl.ds(0, 128)], bufm[sl],
                                  sr[sl]).wait()

        def issue_scat(sl, g):
            pltpu.async_copy(bufm[sl], spm.at[idxd.at[g]], ss[sl], add=True)

        def wait_scat(sl, g):
            pltpu.make_async_copy(bufm[sl], spm.at[idxd.at[g]],
                                  ss[sl]).wait()

        issue_read(0, 0)

        @pl.loop(0, eg, step=2)
        def _(g):
            @pl.when(g >= 2)
            def _():
                wait_scat(1, g - 1)

            @pl.when(g + 1 < eg)
            def _():
                issue_read(1, g + 1)

            wait_read(0)
            issue_scat(0, g)
            wait_scat(0, g)

            @pl.when(g + 2 < eg)
            def _():
                issue_read(0, g + 2)

            @pl.when(g + 1 < eg)
            def _():
                wait_read(1)
                issue_scat(1, g + 1)

        wait_scat(1, eg - 1)
        plsc.subcore_barrier()
        pltpu.sync_copy(spm.at[pl.ds(sid * NROWS_PER_SUB, NROWS_PER_SUB)],
                        o_hbm.at[cid, pl.ds(sid * NROWS_PER_SUB, NROWS_PER_SUB)])

    return k(msgs, dst2d, zeros_tab)


def _dot(a, b):
    return jnp.dot(a, b, preferred_element_type=F32)


def _tc_pool_pos(pos_pad, batch_lanes):
    def body(pos_ref, bl_ref, pooled_ref, counts_ref):
        i = pl.program_id(0)
        brow = bl_ref[0][0:1, :]
        oht = (lax.broadcasted_iota(I32, (128, NBLK), 0) == brow).astype(F32)
        p = _dot(oht, pos_ref[...])
        c = _dot(oht, jnp.ones((NBLK, 128), F32))

        @pl.when(i == 0)
        def _():
            pooled_ref[...] = jnp.zeros_like(pooled_ref)
            counts_ref[...] = jnp.zeros_like(counts_ref)

        pooled_ref[...] += p
        counts_ref[...] += c

    return pl.pallas_call(
        body,
        grid=(NPAD // NBLK,),
        in_specs=[
            pl.BlockSpec((NBLK, 128), lambda i: (i, 0)),
            pl.BlockSpec((1, 8, NBLK), lambda i: (i, 0, 0)),
        ],
        out_specs=[
            pl.BlockSpec((128, 128), lambda i: (0, 0)),
            pl.BlockSpec((128, 128), lambda i: (0, 0)),
        ],
        out_shape=[jax.ShapeDtypeStruct((128, 128), F32),
                   jax.ShapeDtypeStruct((128, 128), F32)],
    )(pos_pad, batch_lanes)


def _tc_locmean(pos_pad, batchcol, pooled, countsm):
    def body(pos_ref, cb_ref, pooled_ref, counts_ref, out_ref):
        cent = pooled_ref[...] / jnp.maximum(counts_ref[...], 1.0)
        oh = ((cb_ref[...] >> 2) ==
              lax.broadcasted_iota(I32, (NBLK, 128), 1)).astype(F32)
        out_ref[...] = pos_ref[...] - _dot(oh, cent)

    return pl.pallas_call(
        body,
        grid=(NPAD // NBLK,),
        in_specs=[
            pl.BlockSpec((NBLK, 128), lambda i: (i, 0)),
            pl.BlockSpec((NBLK, 128), lambda i: (i, 0)),
            pl.BlockSpec((128, 128), lambda i: (0, 0)),
            pl.BlockSpec((128, 128), lambda i: (0, 0)),
        ],
        out_specs=pl.BlockSpec((NBLK, 128), lambda i: (i, 0)),
        out_shape=jax.ShapeDtypeStruct((NPAD, 128), F32),
    )(pos_pad, batchcol, pooled, countsm)


def _tc_embed(g, combo, wemb, bemb, wm1, bm1row, n_real):
    def body(g0_ref, g1_ref, g2_ref, cb_ref, we_ref, be_ref, wm1_ref,
             bm1_ref, x_ref, a_ref, b_ref):
        g0 = g0_ref[...]
        g1 = g1_ref[...]
        g2 = g2_ref[...]

        def emb(i):
            return (_dot(g0, we_ref[i, 0:128, :]) +
                    _dot(g1, we_ref[i, 128:256, :]) +
                    _dot(g2, we_ref[i, 256:384, :]) +
                    be_ref[8 * i:8 * i + 1, :])

        nt = cb_ref[...] & 3
        x = jnp.where(nt == 0, emb(0), jnp.where(nt == 1, emb(1), emb(2)))
        i = pl.program_id(0)
        row = lax.broadcasted_iota(I32, (NBLK, 128), 0) + i * NBLK
        x = jnp.where(row < n_real, x, 0.0)
        x_ref[...] = x
        a_ref[...] = _dot(x, wm1_ref[0:H, :]) + bm1_ref[0:1, :]
        b_ref[...] = _dot(x, wm1_ref[H:2 * H, :])

    nblocks = NPAD // NBLK
    return pl.pallas_call(
        body,
        grid=(nblocks,),
        in_specs=[
            pl.BlockSpec((NBLK, 128), lambda i: (i, 0)),
            pl.BlockSpec((NBLK, 128), lambda i: (i + nblocks, 0)),
            pl.BlockSpec((NBLK, 128), lambda i: (i + 2 * nblocks, 0)),
            pl.BlockSpec((NBLK, 128), lambda i: (i, 0)),
            pl.BlockSpec((3, 384, 128), lambda i: (0, 0, 0)),
            pl.BlockSpec((24, 128), lambda i: (0, 0)),
            pl.BlockSpec((2 * H, H), lambda i: (0, 0)),
            pl.BlockSpec((8, 128), lambda i: (0, 0)),
        ],
        out_specs=[pl.BlockSpec((NBLK, 128), lambda i: (i, 0))] * 3,
        out_shape=[jax.ShapeDtypeStruct((NPAD, 128), F32)] * 3,
    )(g, g, g, combo, wemb, bemb, wm1, bm1row)


def _tc_edge_mlp(grows, wm2, bm2row):
    def body(g_ref, w2_ref, b2_ref, out_ref):
        gsum = jnp.maximum(g_ref[...], 0.0)
        out_ref[...] = jnp.maximum(_dot(gsum, w2_ref[...]) + b2_ref[0:1, :],
                                   0.0)

    ne = grows.shape[0]
    return pl.pallas_call(
        body,
        grid=(ne // EBLK,),
        in_specs=[
            pl.BlockSpec((EBLK, H), lambda i: (i, 0)),
            pl.BlockSpec((H, H), lambda i: (0, 0)),
            pl.BlockSpec((8, 128), lambda i: (0, 0)),
        ],
        out_specs=pl.BlockSpec((EBLK, H), lambda i: (i, 0)),
        out_shape=jax.ShapeDtypeStruct((ne, H), F32),
    )(grows, wm2, bm2row)


def _tc_update(x, aggp0, aggp1, wu1, bu1row, wu2, bu2row, wm1n, bm1nrow,
               n_real):
    def body(x_ref, p_ref, q_ref, wu1_ref, bu1_ref, wu2_ref, bu2_ref,
             wm1_ref, bm1_ref, xo_ref, ao_ref, bo_ref):
        agg = p_ref[0] + p_ref[1] + q_ref[0] + q_ref[1]
        x_in = x_ref[...]
        h = jnp.maximum(_dot(x_in, wu1_ref[0:H, :]) +
                        _dot(agg, wu1_ref[H:2 * H, :]) + bu1_ref[0:1, :], 0.0)
        xn = _dot(h, wu2_ref[...]) + bu2_ref[0:1, :]
        i = pl.program_id(0)
        row = lax.broadcasted_iota(I32, (NBLK, 128), 0) + i * NBLK
        xn = jnp.where(row < n_real, xn, 0.0)
        xo_ref[...] = xn
        ao_ref[...] = _dot(xn, wm1_ref[0:H, :]) + bm1_ref[0:1, :]
        bo_ref[...] = _dot(xn, wm1_ref[H:2 * H, :])

    return pl.pallas_call(
        body,
        grid=(NPAD // NBLK,),
        in_specs=[
            pl.BlockSpec((NBLK, 128), lambda i: (i, 0)),
            pl.BlockSpec((2, NBLK, 128), lambda i: (0, i, 0)),
            pl.BlockSpec((2, NBLK, 128), lambda i: (0, i, 0)),
            pl.BlockSpec((2 * H, H), lambda i: (0, 0)),
            pl.BlockSpec((8, 128), lambda i: (0, 0)),
            pl.BlockSpec((H, H), lambda i: (0, 0)),
            pl.BlockSpec((8, 128), lambda i: (0, 0)),
            pl.BlockSpec((2 * H, H), lambda i: (0, 0)),
            pl.BlockSpec((8, 128), lambda i: (0, 0)),
        ],
        out_specs=[pl.BlockSpec((NBLK, 128), lambda i: (i, 0))] * 3,
        out_shape=[jax.ShapeDtypeStruct((NPAD, 128), F32)] * 3,
    )(x, aggp0, aggp1, wu1, bu1row, wu2, bu2row, wm1n, bm1nrow)


def _tc_update_final(x, aggp0, aggp1, wu1, bu1row, wu2, bu2row, batch_lanes,
                     n_real):
    def body(x_ref, p_ref, q_ref, wu1_ref, bu1_ref, wu2_ref, bu2_ref, bl_ref,
             pooled_ref):
        agg = p_ref[0] + p_ref[1] + q_ref[0] + q_ref[1]
        x_in = x_ref[...]
        h = jnp.maximum(_dot(x_in, wu1_ref[0:H, :]) +
                        _dot(agg, wu1_ref[H:2 * H, :]) + bu1_ref[0:1, :], 0.0)
        xn = _dot(h, wu2_ref[...]) + bu2_ref[0:1, :]
        i = pl.program_id(0)
        row = lax.broadcasted_iota(I32, (NBLK, 128), 0) + i * NBLK
        xn = jnp.where(row < n_real, xn, 0.0)
        brow = bl_ref[0][0:1, :]
        oht = (lax.broadcasted_iota(I32, (128, NBLK), 0) == brow).astype(F32)
        p = _dot(oht, xn)

        @pl.when(i == 0)
        def _():
            pooled_ref[...] = jnp.zeros_like(pooled_ref)

        pooled_ref[...] += p

    return pl.pallas_call(
        body,
        grid=(NPAD // NBLK,),
        in_specs=[
            pl.BlockSpec((NBLK, 128), lambda i: (i, 0)),
            pl.BlockSpec((2, NBLK, 128), lambda i: (0, i, 0)),
            pl.BlockSpec((2, NBLK, 128), lambda i: (0, i, 0)),
            pl.BlockSpec((2 * H, H), lambda i: (0, 0)),
            pl.BlockSpec((8, 128), lambda i: (0, 0)),
            pl.BlockSpec((H, H), lambda i: (0, 0)),
            pl.BlockSpec((8, 128), lambda i: (0, 0)),
            pl.BlockSpec((1, 8, NBLK), lambda i: (i, 0, 0)),
        ],
        out_specs=pl.BlockSpec((128, 128), lambda i: (0, 0)),
        out_shape=jax.ShapeDtypeStruct((128, 128), F32),
    )(x, aggp0, aggp1, wu1, bu1row, wu2, bu2row, batch_lanes)


def _tc_head(pooled, countsm, wp_pad, bprow, ypad, n_graphs):
    def body(pooled_ref, counts_ref, wp_ref, bp_ref, y_ref,
             loss_ref, acc_ref, mloss_ref):
        pm = pooled_ref[...] / jnp.maximum(counts_ref[...], 1.0)
        out = _dot(pm, wp_ref[...]) + bp_ref[0:1, :]
        lane = lax.broadcasted_iota(I32, (128, 128), 1)
        row = lax.broadcasted_iota(I32, (128, 128), 0)
        z = jnp.where(lane < 2, out, -1e30)
        m = jnp.max(z, axis=1, keepdims=True)
        s = jnp.sum(jnp.where(lane < 2, jnp.exp(z - m), 0.0), axis=1,
                    keepdims=True)
        lse = m + jnp.log(s)
        y = y_ref[...]
        lossv = -jnp.sum(jnp.where(lane < 2, y * (out - lse), 0.0), axis=1,
                         keepdims=True)
        o0 = out[:, 0:1]
        o1 = out[:, 1:2]
        y0 = y[:, 0:1]
        y1 = y[:, 1:2]
        accv = ((o1 > o0) == (y1 > y0)).astype(F32)
        loss_ref[...] = jnp.broadcast_to(lossv, (128, 128))
        acc_ref[...] = jnp.broadcast_to(accv, (128, 128))
        total = jnp.sum(jnp.where(row[:, 0:1] < n_graphs, lossv, 0.0))
        mloss_ref[...] = jnp.full((8, 128), total / n_graphs, F32)

    return pl.pallas_call(
        body,
        grid=(1,),
        in_specs=[pl.BlockSpec((128, 128), lambda i: (0, 0))] * 5,
        out_specs=[
            pl.BlockSpec((128, 128), lambda i: (0, 0)),
            pl.BlockSpec((128, 128), lambda i: (0, 0)),
            pl.BlockSpec((8, 128), lambda i: (0, 0)),
        ],
        out_shape=[jax.ShapeDtypeStruct((128, 128), F32),
                   jax.ShapeDtypeStruct((128, 128), F32),
                   jax.ShapeDtypeStruct((8, 128), F32)],
    )(pooled, countsm, wp_pad, bprow, ypad)


def _rowpad(b):
    return jnp.broadcast_to(b[None, :], (8, 128)).astype(F32)


def kernel(pos, y, params, node_types, x_ind, batch, edge_index):
    n = pos.shape[0]
    e = edge_index.shape[1]
    nb = y.shape[0]

    pos_pad = jnp.pad(pos.astype(F32), ((0, NPAD - n), (0, 128 - 3)))
    batch_pad = jnp.concatenate(
        [batch.astype(I32), jnp.full((NPAD - n,), nb, I32)])
    batch_lanes = jnp.broadcast_to(
        batch_pad.reshape(NPAD // NBLK, 1, NBLK), (NPAD // NBLK, 8, NBLK))
    nt_pad = jnp.pad(node_types.astype(I32), (0, NPAD - n))
    combo = jnp.broadcast_to(((batch_pad << 2) | nt_pad)[:, None],
                             (NPAD, 128))

    src = edge_index[0].astype(I32)
    dst = edge_index[1].astype(I32)
    epad_ids = jnp.arange(EPAD - e, dtype=I32)
    src2d = jnp.concatenate([src, epad_ids % n]).reshape(-1, 128)
    dst2d = jnp.concatenate([dst, n + epad_ids % (NPAD - n)]).reshape(-1, 128)

    xif = jnp.pad(x_ind.astype(I32), ((0, NPAD - n), (0, 0))).T.reshape(-1)
    xif2d = jnp.concatenate(
        [xif, jnp.arange(XPAD - 3 * NPAD, dtype=I32) % n]).reshape(-1, 128)

    zeros_tab = jnp.zeros((NPAD, H), F32)

    wemb = jnp.zeros((3, 384, 128), F32)
    for i in range(3):
        wi = params['emb'][i]['W']
        for j in range(i + 1):
            wemb = wemb.at[i, 128 * j:128 * j + 3, :].set(wi[3 * j:3 * j + 3])
    bemb = jnp.zeros((24, 128), F32)
    for i in range(3):
        bemb = bemb.at[8 * i, :].set(params['emb'][i]['b'])

    wp_pad = jnp.zeros((128, 128), F32).at[:, :2].set(params['proj']['W'])
    bprow = jnp.zeros((8, 128), F32).at[0, :2].set(params['proj']['b'])
    bprow = jnp.broadcast_to(bprow[0:1, :], (8, 128))
    ypad = jnp.pad(y.astype(F32), ((0, 128 - nb), (0, 128 - 2)))

    layers = params['layers']

    pooled_pos, countsm = _tc_pool_pos(pos_pad, batch_lanes)
    locmean = _tc_locmean(pos_pad, combo, pooled_pos, countsm)

    g = _sc_gather16(locmean, xif2d)

    lp = layers[0]
    x, ta, tb = _tc_embed(g, combo, wemb, bemb, lp['Wm1'],
                          _rowpad(lp['bm1']), n)

    hg = src2d.shape[0] // 2
    egh = hg // NWORK
    s2d = (src2d[:hg], src2d[hg:])
    d2d = (dst2d[:hg], dst2d[hg:])
    for l in range(3):
        lp = layers[l]
        bm2r = _rowpad(lp['bm2'])
        g0 = _sc_edge_gather(ta, tb, s2d[0], d2d[0], egh)
        g1 = _sc_edge_gather(ta, tb, s2d[1], d2d[1], egh)
        m0 = _tc_edge_mlp(g0, lp['Wm2'], bm2r)
        m1 = _tc_edge_mlp(g1, lp['Wm2'], bm2r)
        aggp0 = _sc_scatter_add(m0, d2d[0], zeros_tab, egh)
        aggp1 = _sc_scatter_add(m1, d2d[1], zeros_tab, egh)
        if l < 2:
            lpn = layers[l + 1]
            x, ta, tb = _tc_update(
                x, aggp0, aggp1, lp['Wu1'], _rowpad(lp['bu1']), lp['Wu2'],
                _rowpad(lp['bu2']), lpn['Wm1'], _rowpad(lpn['bm1']), n)
        else:
            pooled_x = _tc_update_final(
                x, aggp0, aggp1, lp['Wu1'], _rowpad(lp['bu1']), lp['Wu2'],
                _rowpad(lp['bu2']), batch_lanes, n)

    loss128, acc128, mloss = _tc_head(pooled_x, countsm, wp_pad, bprow, ypad,
                                      nb)

    loss = loss128[:nb, 0]
    acc = acc128[:nb, 0]
    backprop_loss = mloss[0, 0].reshape(())
    return backprop_loss, loss, acc


# speedup vs baseline: 1.2130x; 1.0755x over previous
import functools

import jax
import jax.numpy as jnp
from jax import lax
from jax.experimental import pallas as pl
from jax.experimental.pallas import tpu as pltpu
from jax.experimental.pallas import tpu_sc as plsc

F32 = jnp.float32
BF16 = jnp.bfloat16
I32 = jnp.int32

H = 128
NPAD = 10240
EPAD = 327680
XPAD = 32768
NBLK = 2048
EBLK = 4096
NWORK = 32
EG = 80
NROWS_PER_SUB = NPAD // 16

@functools.cache
def _mesh():
    return plsc.VectorSubcoreMesh(core_axis_name="c", subcore_axis_name="s",
                                  num_cores=2, num_subcores=16)


def _sc_gather16(table, idx2d):
    @functools.partial(
        pl.kernel,
        out_type=jax.ShapeDtypeStruct((XPAD, 128), F32),
        mesh=_mesh(),
        scratch_types=[
            pltpu.VMEM((8, 128), I32),
            pltpu.VMEM((128, 128), F32),
            pltpu.SemaphoreType.DMA,
        ],
    )
    def k(tab_hbm, idx_hbm, out_hbm, idx_v, buf, sem):
        wid = lax.axis_index("s") * 2 + lax.axis_index("c")
        pltpu.sync_copy(idx_hbm.at[pl.ds(wid * 8, 8)], idx_v)

        @pl.loop(0, 8)
        def _(g):
            pltpu.async_copy(tab_hbm.at[idx_v.at[g]], buf, sem).wait()
            pltpu.sync_copy(buf, out_hbm.at[pl.ds(wid * 1024 + g * 128, 128)])

    return k(table, idx2d)


def _sc_edge_gather(tab_a, tab_b, src2d, dst2d, eg):
    ne = NWORK * eg * 128
    @functools.partial(
        pl.kernel,
        out_type=jax.ShapeDtypeStruct((ne, H), F32),
        mesh=_mesh(),
        scratch_types=[
            pltpu.VMEM((eg, 128), I32),
            pltpu.VMEM((eg, 128), I32),
            pltpu.VMEM((128, H), F32),
            pltpu.VMEM((128, H), F32),
            pltpu.VMEM_SHARED((NPAD, H), F32),
        ] + [pltpu.SemaphoreType.DMA] * 4,
    )
    def k(a_hbm, b_hbm, s_hbm, d_hbm, o_hbm,
          idxa, idxb, buf0, buf1, spa,
          sg0, sg1, sw0, sw1):
        sid = lax.axis_index("s")
        wid = sid * 2 + lax.axis_index("c")
        pltpu.sync_copy(a_hbm.at[pl.ds(sid * NROWS_PER_SUB, NROWS_PER_SUB)],
                        spa.at[pl.ds(sid * NROWS_PER_SUB, NROWS_PER_SUB)])
        pltpu.sync_copy(s_hbm.at[pl.ds(wid * eg, eg)], idxa)
        pltpu.sync_copy(d_hbm.at[pl.ds(wid * eg, eg)], idxb)
        plsc.subcore_barrier()
        bufs = (buf0, buf1)
        sg = (sg0, sg1)
        sw = (sw0, sw1)

        def issue_ga(sl, g):
            pltpu.async_copy(b_hbm.at[idxb.at[g]], bufs[sl], sg[sl])

        def wait_ga(sl, g):
            pltpu.make_async_copy(b_hbm.at[idxb.at[g]], bufs[sl],
                                  sg[sl]).wait()

        def issue_gb(sl, g):
            pltpu.async_copy(spa.at[idxa.at[g]], bufs[sl], sg[sl],
                             add=True)

        def wait_gb(sl, g):
            pltpu.make_async_copy(spa.at[idxa.at[g]], bufs[sl],
                                  sg[sl]).wait()

        def issue_w(sl, g):
            row0 = wid * (eg * 128) + g * 128
            pltpu.async_copy(bufs[sl], o_hbm.at[pl.ds(row0, 128)], sw[sl])

        def wait_w(sl):
            pltpu.make_async_copy(bufs[sl], o_hbm.at[pl.ds(0, 128)],
                                  sw[sl]).wait()

        issue_ga(0, 0)

        @pl.loop(0, eg, step=2)
        def _(g):
            @pl.when(g >= 2)
            def _():
                wait_w(1)

            @pl.when(g + 1 < eg)
            def _():
                issue_ga(1, g + 1)

            wait_ga(0, g)
            issue_gb(0, g)
            wait_gb(0, g)
            issue_w(0, g)
            wait_w(0)

            @pl.when(g + 2 < eg)
            def _():
                issue_ga(0, g + 2)

            @pl.when(g + 1 < eg)
            def _():
                wait_ga(1, g + 1)
                issue_gb(1, g + 1)
                wait_gb(1, g + 1)
                issue_w(1, g + 1)

        wait_w(1)

    return k(tab_a, tab_b, src2d, dst2d)


def _sc_scatter_add(msgs, dst2d, zeros_tab, eg):
    @functools.partial(
        pl.kernel,
        out_type=jax.ShapeDtypeStruct((2, NPAD, H), F32),
        mesh=_mesh(),
        scratch_types=[
            pltpu.VMEM((eg, 128), I32),
            pltpu.VMEM((128, H), F32),
            pltpu.VMEM((128, H), F32),
            pltpu.VMEM_SHARED((NPAD, H), F32),
        ] + [pltpu.SemaphoreType.DMA] * 4,
    )
    def k(m_hbm, d_hbm, z_hbm, o_hbm, idxd, bufm0, bufm1, spm,
          sr0, sr1, ss0, ss1):
        cid = lax.axis_index("c")
        sid = lax.axis_index("s")
        wid = sid * 2 + cid
        pltpu.sync_copy(z_hbm.at[pl.ds(sid * NROWS_PER_SUB, NROWS_PER_SUB)],
                        spm.at[pl.ds(sid * NROWS_PER_SUB, NROWS_PER_SUB)])
        plsc.subcore_barrier()
        pltpu.sync_copy(d_hbm.at[pl.ds(wid * eg, eg)], idxd)
        bufm = (bufm0, bufm1)
        sr = (sr0, sr1)
        ss = (ss0, ss1)

        def issue_read(sl, g):
            pltpu.async_copy(m_hbm.at[pl.ds(wid * (eg * 128) + g * 128, 128)],
                             bufm[sl], sr[sl])

        def wait_read(sl):
            pltpu.make_async_copy(m_hbm.at[pl.ds(0, 128)], bufm[sl],
                                  sr[sl]).wait()

        def issue_scat(sl, g):
            pltpu.async_copy(bufm[sl], spm.at[idxd.at[g]], ss[sl], add=True)

        def wait_scat(sl, g):
            pltpu.make_async_copy(bufm[sl], spm.at[idxd.at[g]],
                                  ss[sl]).wait()

        issue_read(0, 0)

        @pl.loop(0, eg, step=2)
        def _(g):
            @pl.when(g >= 2)
            def _():
                wait_scat(1, g - 1)

            @pl.when(g + 1 < eg)
            def _():
                issue_read(1, g + 1)

            wait_read(0)
            issue_scat(0, g)
            wait_scat(0, g)

            @pl.when(g + 2 < eg)
            def _():
                issue_read(0, g + 2)

            @pl.when(g + 1 < eg)
            def _():
                wait_read(1)
                issue_scat(1, g + 1)

        wait_scat(1, eg - 1)
        plsc.subcore_barrier()
        pltpu.sync_copy(spm.at[pl.ds(sid * NROWS_PER_SUB, NROWS_PER_SUB)],
                        o_hbm.at[cid, pl.ds(sid * NROWS_PER_SUB, NROWS_PER_SUB)])

    return k(msgs, dst2d, zeros_tab)


def _dot(a, b):
    return jnp.dot(a, b, preferred_element_type=F32)


def _tc_pool_pos(pos_pad, batch_lanes):
    def body(pos_ref, bl_ref, pooled_ref, counts_ref):
        i = pl.program_id(0)
        brow = bl_ref[0][0:1, :]
        oht = (lax.broadcasted_iota(I32, (128, NBLK), 0) == brow).astype(F32)
        p = _dot(oht, pos_ref[...])
        c = _dot(oht, jnp.ones((NBLK, 128), F32))

        @pl.when(i == 0)
        def _():
            pooled_ref[...] = jnp.zeros_like(pooled_ref)
            counts_ref[...] = jnp.zeros_like(counts_ref)

        pooled_ref[...] += p
        counts_ref[...] += c

    return pl.pallas_call(
        body,
        grid=(NPAD // NBLK,),
        in_specs=[
            pl.BlockSpec((NBLK, 128), lambda i: (i, 0)),
            pl.BlockSpec((1, 8, NBLK), lambda i: (i, 0, 0)),
        ],
        out_specs=[
            pl.BlockSpec((128, 128), lambda i: (0, 0)),
            pl.BlockSpec((128, 128), lambda i: (0, 0)),
        ],
        out_shape=[jax.ShapeDtypeStruct((128, 128), F32),
                   jax.ShapeDtypeStruct((128, 128), F32)],
    )(pos_pad, batch_lanes)


def _tc_locmean(pos_pad, batchcol, pooled, countsm):
    def body(pos_ref, cb_ref, pooled_ref, counts_ref, out_ref):
        cent = pooled_ref[...] / jnp.maximum(counts_ref[...], 1.0)
        oh = ((cb_ref[...] >> 2) ==
              lax.broadcasted_iota(I32, (NBLK, 128), 1)).astype(F32)
        out_ref[...] = pos_ref[...] - _dot(oh, cent)

    return pl.pallas_call(
        body,
        grid=(NPAD // NBLK,),
        in_specs=[
            pl.BlockSpec((NBLK, 128), lambda i: (i, 0)),
            pl.BlockSpec((NBLK, 128), lambda i: (i, 0)),
            pl.BlockSpec((128, 128), lambda i: (0, 0)),
            pl.BlockSpec((128, 128), lambda i: (0, 0)),
        ],
        out_specs=pl.BlockSpec((NBLK, 128), lambda i: (i, 0)),
        out_shape=jax.ShapeDtypeStruct((NPAD, 128), F32),
    )(pos_pad, batchcol, pooled, countsm)


def _tc_embed(g, combo, wemb, bemb, wm1, bm1row, n_real):
    def body(g0_ref, g1_ref, g2_ref, cb_ref, we_ref, be_ref, wm1_ref,
             bm1_ref, x_ref, a_ref, b_ref):
        g0 = g0_ref[...]
        g1 = g1_ref[...]
        g2 = g2_ref[...]

        def emb(i):
            return (_dot(g0, we_ref[i, 0:128, :]) +
                    _dot(g1, we_ref[i, 128:256, :]) +
                    _dot(g2, we_ref[i, 256:384, :]) +
                    be_ref[8 * i:8 * i + 1, :])

        nt = cb_ref[...] & 3
        x = jnp.where(nt == 0, emb(0), jnp.where(nt == 1, emb(1), emb(2)))
        i = pl.program_id(0)
        row = lax.broadcasted_iota(I32, (NBLK, 128), 0) + i * NBLK
        x = jnp.where(row < n_real, x, 0.0)
        x_ref[...] = x
        a_ref[...] = _dot(x, wm1_ref[0:H, :]) + bm1_ref[0:1, :]
        b_ref[...] = _dot(x, wm1_ref[H:2 * H, :])

    nblocks = NPAD // NBLK
    return pl.pallas_call(
        body,
        grid=(nblocks,),
        in_specs=[
            pl.BlockSpec((NBLK, 128), lambda i: (i, 0)),
            pl.BlockSpec((NBLK, 128), lambda i: (i + nblocks, 0)),
            pl.BlockSpec((NBLK, 128), lambda i: (i + 2 * nblocks, 0)),
            pl.BlockSpec((NBLK, 128), lambda i: (i, 0)),
            pl.BlockSpec((3, 384, 128), lambda i: (0, 0, 0)),
            pl.BlockSpec((24, 128), lambda i: (0, 0)),
            pl.BlockSpec((2 * H, H), lambda i: (0, 0)),
            pl.BlockSpec((8, 128), lambda i: (0, 0)),
        ],
        out_specs=[pl.BlockSpec((NBLK, 128), lambda i: (i, 0))] * 3,
        out_shape=[jax.ShapeDtypeStruct((NPAD, 128), F32)] * 3,
    )(g, g, g, combo, wemb, bemb, wm1, bm1row)


def _tc_edge_mlp(grows, wm2, bm2row):
    def body(g_ref, w2_ref, b2_ref, out_ref):
        gsum = jnp.maximum(g_ref[...], 0.0)
        out_ref[...] = jnp.maximum(_dot(gsum, w2_ref[...]) + b2_ref[0:1, :],
                                   0.0)

    ne = grows.shape[0]
    return pl.pallas_call(
        body,
        grid=(ne // EBLK,),
        in_specs=[
            pl.BlockSpec((EBLK, H), lambda i: (i, 0)),
            pl.BlockSpec((H, H), lambda i: (0, 0)),
            pl.BlockSpec((8, 128), lambda i: (0, 0)),
        ],
        out_specs=pl.BlockSpec((EBLK, H), lambda i: (i, 0)),
        out_shape=jax.ShapeDtypeStruct((ne, H), F32),
    )(grows, wm2, bm2row)


def _tc_update(x, aggp0, aggp1, wu1, bu1row, wu2, bu2row, wm1n, bm1nrow,
               n_real):
    def body(x_ref, p_ref, q_ref, wu1_ref, bu1_ref, wu2_ref, bu2_ref,
             wm1_ref, bm1_ref, xo_ref, ao_ref, bo_ref):
        agg = p_ref[0] + p_ref[1] + q_ref[0] + q_ref[1]
        x_in = x_ref[...]
        h = jnp.maximum(_dot(x_in, wu1_ref[0:H, :]) +
                        _dot(agg, wu1_ref[H:2 * H, :]) + bu1_ref[0:1, :], 0.0)
        xn = _dot(h, wu2_ref[...]) + bu2_ref[0:1, :]
        i = pl.program_id(0)
        row = lax.broadcasted_iota(I32, (NBLK, 128), 0) + i * NBLK
        xn = jnp.where(row < n_real, xn, 0.0)
        xo_ref[...] = xn
        ao_ref[...] = _dot(xn, wm1_ref[0:H, :]) + bm1_ref[0:1, :]
        bo_ref[...] = _dot(xn, wm1_ref[H:2 * H, :])

    return pl.pallas_call(
        body,
        grid=(NPAD // NBLK,),
        in_specs=[
            pl.BlockSpec((NBLK, 128), lambda i: (i, 0)),
            pl.BlockSpec((2, NBLK, 128), lambda i: (0, i, 0)),
            pl.BlockSpec((2, NBLK, 128), lambda i: (0, i, 0)),
            pl.BlockSpec((2 * H, H), lambda i: (0, 0)),
            pl.BlockSpec((8, 128), lambda i: (0, 0)),
            pl.BlockSpec((H, H), lambda i: (0, 0)),
            pl.BlockSpec((8, 128), lambda i: (0, 0)),
            pl.BlockSpec((2 * H, H), lambda i: (0, 0)),
            pl.BlockSpec((8, 128), lambda i: (0, 0)),
        ],
        out_specs=[pl.BlockSpec((NBLK, 128), lambda i: (i, 0))] * 3,
        out_shape=[jax.ShapeDtypeStruct((NPAD, 128), F32)] * 3,
    )(x, aggp0, aggp1, wu1, bu1row, wu2, bu2row, wm1n, bm1nrow)


def _tc_update_final(x, aggp0, aggp1, wu1, bu1row, wu2, bu2row, batch_lanes,
                     n_real):
    def body(x_ref, p_ref, q_ref, wu1_ref, bu1_ref, wu2_ref, bu2_ref, bl_ref,
             pooled_ref):
        agg = p_ref[0] + p_ref[1] + q_ref[0] + q_ref[1]
        x_in = x_ref[...]
        h = jnp.maximum(_dot(x_in, wu1_ref[0:H, :]) +
                        _dot(agg, wu1_ref[H:2 * H, :]) + bu1_ref[0:1, :], 0.0)
        xn = _dot(h, wu2_ref[...]) + bu2_ref[0:1, :]
        i = pl.program_id(0)
        row = lax.broadcasted_iota(I32, (NBLK, 128), 0) + i * NBLK
        xn = jnp.where(row < n_real, xn, 0.0)
        brow = bl_ref[0][0:1, :]
        oht = (lax.broadcasted_iota(I32, (128, NBLK), 0) == brow).astype(F32)
        p = _dot(oht, xn)

        @pl.when(i == 0)
        def _():
            pooled_ref[...] = jnp.zeros_like(pooled_ref)

        pooled_ref[...] += p

    return pl.pallas_call(
        body,
        grid=(NPAD // NBLK,),
        in_specs=[
            pl.BlockSpec((NBLK, 128), lambda i: (i, 0)),
            pl.BlockSpec((2, NBLK, 128), lambda i: (0, i, 0)),
            pl.BlockSpec((2, NBLK, 128), lambda i: (0, i, 0)),
            pl.BlockSpec((2 * H, H), lambda i: (0, 0)),
            pl.BlockSpec((8, 128), lambda i: (0, 0)),
            pl.BlockSpec((H, H), lambda i: (0, 0)),
            pl.BlockSpec((8, 128), lambda i: (0, 0)),
            pl.BlockSpec((1, 8, NBLK), lambda i: (i, 0, 0)),
        ],
        out_specs=pl.BlockSpec((128, 128), lambda i: (0, 0)),
        out_shape=jax.ShapeDtypeStruct((128, 128), F32),
    )(x, aggp0, aggp1, wu1, bu1row, wu2, bu2row, batch_lanes)


def _tc_head(pooled, countsm, wp_pad, bprow, ypad, n_graphs):
    def body(pooled_ref, counts_ref, wp_ref, bp_ref, y_ref,
             loss_ref, acc_ref, mloss_ref):
        pm = pooled_ref[...] / jnp.maximum(counts_ref[...], 1.0)
        out = _dot(pm, wp_ref[...]) + bp_ref[0:1, :]
        lane = lax.broadcasted_iota(I32, (128, 128), 1)
        row = lax.broadcasted_iota(I32, (128, 128), 0)
        z = jnp.where(lane < 2, out, -1e30)
        m = jnp.max(z, axis=1, keepdims=True)
        s = jnp.sum(jnp.where(lane < 2, jnp.exp(z - m), 0.0), axis=1,
                    keepdims=True)
        lse = m + jnp.log(s)
        y = y_ref[...]
        lossv = -jnp.sum(jnp.where(lane < 2, y * (out - lse), 0.0), axis=1,
                         keepdims=True)
        o0 = out[:, 0:1]
        o1 = out[:, 1:2]
        y0 = y[:, 0:1]
        y1 = y[:, 1:2]
        accv = ((o1 > o0) == (y1 > y0)).astype(F32)
        loss_ref[...] = jnp.broadcast_to(lossv, (128, 128))
        acc_ref[...] = jnp.broadcast_to(accv, (128, 128))
        total = jnp.sum(jnp.where(row[:, 0:1] < n_graphs, lossv, 0.0))
        mloss_ref[...] = jnp.full((8, 128), total / n_graphs, F32)

    return pl.pallas_call(
        body,
        grid=(1,),
        in_specs=[pl.BlockSpec((128, 128), lambda i: (0, 0))] * 5,
        out_specs=[
            pl.BlockSpec((128, 128), lambda i: (0, 0)),
            pl.BlockSpec((128, 128), lambda i: (0, 0)),
            pl.BlockSpec((8, 128), lambda i: (0, 0)),
        ],
        out_shape=[jax.ShapeDtypeStruct((128, 128), F32),
                   jax.ShapeDtypeStruct((128, 128), F32),
                   jax.ShapeDtypeStruct((8, 128), F32)],
    )(pooled, countsm, wp_pad, bprow, ypad)


def _rowpad(b):
    return jnp.broadcast_to(b[None, :], (8, 128)).astype(F32)


def kernel(pos, y, params, node_types, x_ind, batch, edge_index):
    n = pos.shape[0]
    e = edge_index.shape[1]
    nb = y.shape[0]

    pos_pad = jnp.pad(pos.astype(F32), ((0, NPAD - n), (0, 128 - 3)))
    batch_pad = jnp.concatenate(
        [batch.astype(I32), jnp.full((NPAD - n,), nb, I32)])
    batch_lanes = jnp.broadcast_to(
        batch_pad.reshape(NPAD // NBLK, 1, NBLK), (NPAD // NBLK, 8, NBLK))
    nt_pad = jnp.pad(node_types.astype(I32), (0, NPAD - n))
    combo = jnp.broadcast_to(((batch_pad << 2) | nt_pad)[:, None],
                             (NPAD, 128))

    src = edge_index[0].astype(I32)
    dst = edge_index[1].astype(I32)
    epad_ids = jnp.arange(EPAD - e, dtype=I32)
    src2d = jnp.concatenate([src, epad_ids % n]).reshape(-1, 128)
    dst2d = jnp.concatenate([dst, n + epad_ids % (NPAD - n)]).reshape(-1, 128)

    xif = jnp.pad(x_ind.astype(I32), ((0, NPAD - n), (0, 0))).T.reshape(-1)
    xif2d = jnp.concatenate(
        [xif, jnp.arange(XPAD - 3 * NPAD, dtype=I32) % n]).reshape(-1, 128)

    zeros_tab = jnp.zeros((NPAD, H), F32)

    wemb = jnp.zeros((3, 384, 128), F32)
    for i in range(3):
        wi = params['emb'][i]['W']
        for j in range(i + 1):
            wemb = wemb.at[i, 128 * j:128 * j + 3, :].set(wi[3 * j:3 * j + 3])
    bemb = jnp.zeros((24, 128), F32)
    for i in range(3):
        bemb = bemb.at[8 * i, :].set(params['emb'][i]['b'])

    wp_pad = jnp.zeros((128, 128), F32).at[:, :2].set(params['proj']['W'])
    bprow = jnp.zeros((8, 128), F32).at[0, :2].set(params['proj']['b'])
    bprow = jnp.broadcast_to(bprow[0:1, :], (8, 128))
    ypad = jnp.pad(y.astype(F32), ((0, 128 - nb), (0, 128 - 2)))

    layers = params['layers']

    pooled_pos, countsm = _tc_pool_pos(pos_pad, batch_lanes)
    locmean = _tc_locmean(pos_pad, combo, pooled_pos, countsm)

    g = _sc_gather16(locmean, xif2d)

    lp = layers[0]
    x, ta, tb = _tc_embed(g, combo, wemb, bemb, lp['Wm1'],
                          _rowpad(lp['bm1']), n)

    hg = src2d.shape[0] // 2
    egh = hg // NWORK
    s2d = (src2d[:hg], src2d[hg:])
    d2d = (dst2d[:hg], dst2d[hg:])
    for l in range(3):
        lp = layers[l]
        bm2r = _rowpad(lp['bm2'])
        g0 = _sc_edge_gather(ta, tb, s2d[0], d2d[0], egh)
        g1 = _sc_edge_gather(ta, tb, s2d[1], d2d[1], egh)
        m0 = _tc_edge_mlp(g0, lp['Wm2'], bm2r)
        m1 = _tc_edge_mlp(g1, lp['Wm2'], bm2r)
        aggp0 = _sc_scatter_add(m0, d2d[0], zeros_tab, egh)
        aggp1 = _sc_scatter_add(m1, d2d[1], zeros_tab, egh)
        if l < 2:
            lpn = layers[l + 1]
            x, ta, tb = _tc_update(
                x, aggp0, aggp1, lp['Wu1'], _rowpad(lp['bu1']), lp['Wu2'],
                _rowpad(lp['bu2']), lpn['Wm1'], _rowpad(lpn['bm1']), n)
        else:
            pooled_x = _tc_update_final(
                x, aggp0, aggp1, lp['Wu1'], _rowpad(lp['bu1']), lp['Wu2'],
                _rowpad(lp['bu2']), batch_lanes, n)

    loss128, acc128, mloss = _tc_head(pooled_x, countsm, wp_pad, bprow, ypad,
                                      nb)

    loss = loss128[:nb, 0]
    acc = acc128[:nb, 0]
    backprop_loss = mloss[0, 0].reshape(())
    return backprop_loss, loss, acc
